# Initial kernel scaffold; baseline (speedup 1.0000x reference)
import functools

import jax
import jax.numpy as jnp
from jax import lax
from jax.experimental import pallas as pl
from jax.experimental.pallas import tpu as pltpu
from jax.experimental.pallas import tpu_sc as plsc

_N = 10000
_E = 160000
_D = 256
_H = 16
_DH = 16


def _qkv_body(h_ref, wq_ref, wk_ref, wv_ref, q_ref, k_ref, v_ref):
    hb = h_ref[...]
    q_ref[...] = jnp.dot(hb, wq_ref[...], preferred_element_type=jnp.float32)
    k_ref[...] = jnp.dot(hb, wk_ref[...], preferred_element_type=jnp.float32)
    v_ref[...] = jnp.dot(hb, wv_ref[...], preferred_element_type=jnp.float32)


def _qkv(h, Wq, Wk, Wv):
    BN = 1000
    mat = pl.BlockSpec((_D, _D), lambda i: (0, 0))
    blk = pl.BlockSpec((BN, _D), lambda i: (i, 0))
    return pl.pallas_call(
        _qkv_body,
        grid=(_N // BN,),
        in_specs=[blk, mat, mat, mat],
        out_specs=[blk, blk, blk],
        out_shape=[jax.ShapeDtypeStruct((_N, _D), jnp.float32)] * 3,
    )(h, Wq, Wk, Wv)


def _edge1_body(e_ref, ks_ref, qd_ref, vs_ref, we_ref, woe_ref, boe_ref,
                hsum_ref, hexp_ref, eo_ref, wv_ref, sexp_ref, stats_ref):
    i = pl.program_id(0)
    eb = e_ref[...]
    pe = jnp.dot(eb, we_ref[...], preferred_element_type=jnp.float32)
    score = ks_ref[...] * qd_ref[...] * (pe * (1.0 / 4.0))
    hs = jnp.dot(score, hsum_ref[...], preferred_element_type=jnp.float32)
    sexp = jnp.exp(jnp.clip(hs, -5.0, 5.0))
    sexp_ref[...] = sexp
    eo = eb + jnp.dot(score, woe_ref[...],
                      preferred_element_type=jnp.float32) + boe_ref[...]
    eo_ref[...] = eo
    wv_ref[...] = vs_ref[...] * jnp.dot(sexp, hexp_ref[...],
                                        preferred_element_type=jnp.float32)

    @pl.when(i == 0)
    def _():
        stats_ref[...] = jnp.zeros_like(stats_ref)

    stats_ref[0:1, :] += jnp.sum(eo, axis=0, keepdims=True)
    stats_ref[1:2, :] += jnp.sum(eo * eo, axis=0, keepdims=True)


def _edge1(e, ksrc, qdst, vsrc, We, Wo_e, bo_e, hsum, hexp):
    BE = 1600
    blk = pl.BlockSpec((BE, _D), lambda i: (i, 0))
    mat = pl.BlockSpec((_D, _D), lambda i: (0, 0))
    return pl.pallas_call(
        _edge1_body,
        grid=(_E // BE,),
        in_specs=[blk, blk, blk, blk, mat, mat,
                  pl.BlockSpec((1, _D), lambda i: (0, 0)),
                  pl.BlockSpec((_D, _H), lambda i: (0, 0)),
                  pl.BlockSpec((_H, _D), lambda i: (0, 0))],
        out_specs=[blk, blk,
                   pl.BlockSpec((BE, _H), lambda i: (i, 0)),
                   pl.BlockSpec((8, _D), lambda i: (0, 0))],
        out_shape=[jax.ShapeDtypeStruct((_E, _D), jnp.float32),
                   jax.ShapeDtypeStruct((_E, _D), jnp.float32),
                   jax.ShapeDtypeStruct((_E, _H), jnp.float32),
                   jax.ShapeDtypeStruct((8, _D), jnp.float32)],
    )(e, ksrc, qdst, vsrc, We, Wo_e, bo_e, hsum, hexp)


def _node1_body(wv_ref, z_ref, h_ref, wo_ref, bo_ref, hexp_ref,
                ho_ref, stats_ref):
    i = pl.program_id(0)
    zrep = jnp.dot(z_ref[...], hexp_ref[...],
                   preferred_element_type=jnp.float32)
    hattn = wv_ref[...] / (zrep + 1e-6)
    ho = h_ref[...] + jnp.dot(hattn, wo_ref[...],
                              preferred_element_type=jnp.float32) + bo_ref[...]
    ho_ref[...] = ho

    @pl.when(i == 0)
    def _():
        stats_ref[...] = jnp.zeros_like(stats_ref)

    stats_ref[0:1, :] += jnp.sum(ho, axis=0, keepdims=True)
    stats_ref[1:2, :] += jnp.sum(ho * ho, axis=0, keepdims=True)


def _node1(wV, z, h, Wo_h, bo_h, hexp):
    BN = 1000
    blk = pl.BlockSpec((BN, _D), lambda i: (i, 0))
    return pl.pallas_call(
        _node1_body,
        grid=(_N // BN,),
        in_specs=[blk, pl.BlockSpec((BN, _H), lambda i: (i, 0)), blk,
                  pl.BlockSpec((_D, _D), lambda i: (0, 0)),
                  pl.BlockSpec((1, _D), lambda i: (0, 0)),
                  pl.BlockSpec((_H, _D), lambda i: (0, 0))],
        out_specs=[blk, pl.BlockSpec((8, _D), lambda i: (0, 0))],
        out_shape=[jax.ShapeDtypeStruct((_N, _D), jnp.float32),
                   jax.ShapeDtypeStruct((8, _D), jnp.float32)],
    )(wV, z, h, Wo_h, bo_h, hexp)


def _ffn_body(count, x_ref, stats_ref, w1_ref, b1_ref, w2_ref, b2_ref,
              out_ref, stats2_ref):
    i = pl.program_id(0)
    mu = stats_ref[0:1, :] * (1.0 / count)
    var = stats_ref[1:2, :] * (1.0 / count) - mu * mu
    inv = lax.rsqrt(var + 1e-5)
    xb = (x_ref[...] - mu) * inv
    u = jnp.maximum(jnp.dot(xb, w1_ref[...],
                            preferred_element_type=jnp.float32) + b1_ref[...],
                    0.0)
    pre2 = xb + jnp.dot(u, w2_ref[...],
                        preferred_element_type=jnp.float32) + b2_ref[...]
    out_ref[...] = pre2

    @pl.when(i == 0)
    def _():
        stats2_ref[...] = jnp.zeros_like(stats2_ref)

    stats2_ref[0:1, :] += jnp.sum(pre2, axis=0, keepdims=True)
    stats2_ref[1:2, :] += jnp.sum(pre2 * pre2, axis=0, keepdims=True)


def _ffn(x, stats, count, W1, b1, W2, b2, bx):
    rows = x.shape[0]
    blk = pl.BlockSpec((bx, _D), lambda i: (i, 0))
    return pl.pallas_call(
        functools.partial(_ffn_body, float(count)),
        grid=(rows // bx,),
        in_specs=[blk, pl.BlockSpec((8, _D), lambda i: (0, 0)),
                  pl.BlockSpec((_D, 2 * _D), lambda i: (0, 0)),
                  pl.BlockSpec((1, 2 * _D), lambda i: (0, 0)),
                  pl.BlockSpec((2 * _D, _D), lambda i: (0, 0)),
                  pl.BlockSpec((1, _D), lambda i: (0, 0))],
        out_specs=[blk, pl.BlockSpec((8, _D), lambda i: (0, 0))],
        out_shape=[jax.ShapeDtypeStruct((rows, _D), jnp.float32),
                   jax.ShapeDtypeStruct((8, _D), jnp.float32)],
    )(x, stats, W1, b1, W2, b2)


def _norm_body(count, x_ref, stats_ref, out_ref):
    mu = stats_ref[0:1, :] * (1.0 / count)
    var = stats_ref[1:2, :] * (1.0 / count) - mu * mu
    inv = lax.rsqrt(var + 1e-5)
    out_ref[...] = (x_ref[...] - mu) * inv


def _norm(x, stats, count, bx):
    rows = x.shape[0]
    blk = pl.BlockSpec((bx, _D), lambda i: (i, 0))
    return pl.pallas_call(
        functools.partial(_norm_body, float(count)),
        grid=(rows // bx,),
        in_specs=[blk, pl.BlockSpec((8, _D), lambda i: (0, 0))],
        out_specs=blk,
        out_shape=jax.ShapeDtypeStruct((rows, _D), jnp.float32),
    )(x, stats)


def _gather3(ktab, qtab, vtab, src, dst):
    NW = 32
    per_w = _E // NW
    CH = 128
    nfull = per_w // CH
    tail = per_w - nfull * CH
    mesh = plsc.VectorSubcoreMesh(core_axis_name="c", subcore_axis_name="s")

    @functools.partial(
        pl.kernel,
        out_type=[jax.ShapeDtypeStruct((_E, _D), jnp.float32)] * 3,
        mesh=mesh,
        scratch_types=[
            pltpu.VMEM((CH,), jnp.int32),
            pltpu.VMEM((CH,), jnp.int32),
            pltpu.VMEM((CH, _D), jnp.float32),
            pltpu.VMEM((tail,), jnp.int32),
            pltpu.VMEM((tail,), jnp.int32),
            pltpu.VMEM((tail, _D), jnp.float32),
            pltpu.SemaphoreType.DMA,
        ])
    def kk(k_hbm, q_hbm, v_hbm, src_hbm, dst_hbm, ks_hbm, qd_hbm, vs_hbm,
           idxs, idxd, rows, tidxs, tidxd, trows, sem):
        c = lax.axis_index("c")
        s = lax.axis_index("s")
        base0 = (s * 2 + c) * per_w

        def chunk(i, _):
            base = base0 + i * CH
            pltpu.sync_copy(src_hbm.at[pl.ds(base, CH)], idxs)
            pltpu.sync_copy(dst_hbm.at[pl.ds(base, CH)], idxd)
            pltpu.async_copy(k_hbm.at[idxs], rows, sem).wait()
            pltpu.sync_copy(rows, ks_hbm.at[pl.ds(base, CH)])
            pltpu.async_copy(q_hbm.at[idxd], rows, sem).wait()
            pltpu.sync_copy(rows, qd_hbm.at[pl.ds(base, CH)])
            pltpu.async_copy(v_hbm.at[idxs], rows, sem).wait()
            pltpu.sync_copy(rows, vs_hbm.at[pl.ds(base, CH)])
            return 0

        lax.fori_loop(0, nfull, chunk, 0)
        base = base0 + nfull * CH
        pltpu.sync_copy(src_hbm.at[pl.ds(base, tail)], tidxs)
        pltpu.sync_copy(dst_hbm.at[pl.ds(base, tail)], tidxd)
        pltpu.async_copy(k_hbm.at[tidxs], trows, sem).wait()
        pltpu.sync_copy(trows, ks_hbm.at[pl.ds(base, tail)])
        pltpu.async_copy(q_hbm.at[tidxd], trows, sem).wait()
        pltpu.sync_copy(trows, qd_hbm.at[pl.ds(base, tail)])
        pltpu.async_copy(v_hbm.at[tidxs], trows, sem).wait()
        pltpu.sync_copy(trows, vs_hbm.at[pl.ds(base, tail)])

    return kk(ktab, qtab, vtab, src, dst)


def _scatter(wv, sexp, dst):
    NSUB = 16
    per_s = _E // NSUB
    CH = 128
    nfull = per_s // CH
    tail = per_s - nfull * CH
    RPS = _N // NSUB
    ZR = 125
    mesh = plsc.VectorSubcoreMesh(core_axis_name="c", subcore_axis_name="s")

    @functools.partial(
        pl.kernel,
        out_type=[jax.ShapeDtypeStruct((_N, _D), jnp.float32),
                  jax.ShapeDtypeStruct((_N, _H), jnp.float32)],
        mesh=mesh,
        scratch_types=[
            pltpu.VMEM_SHARED((_N, 128), jnp.float32),
            pltpu.VMEM_SHARED((_N, _H), jnp.float32),
            pltpu.VMEM((ZR, 128), jnp.float32),
            pltpu.VMEM((ZR, _H), jnp.float32),
            pltpu.VMEM((CH, 128), jnp.float32),
            pltpu.VMEM((CH, _H), jnp.float32),
            pltpu.VMEM((CH,), jnp.int32),
            pltpu.VMEM((tail, 128), jnp.float32),
            pltpu.VMEM((tail, _H), jnp.float32),
            pltpu.VMEM((tail,), jnp.int32),
        ])
    def kk(wv_hbm, sexp_hbm, dst_hbm, out_hbm, outz_hbm,
           acc, accz, zbuf, zbufz, buf, bufz, idxb, tbuf, tbufz, tidxb):
        c = lax.axis_index("c")
        s = lax.axis_index("s")
        coff = c * 128

        def zrow(i, _):
            def zlane(j, _):
                zbuf[i, pl.ds(j * 16, 16)] = jnp.zeros((16,), jnp.float32)
                return 0
            lax.fori_loop(0, 8, zlane, 0)
            zbufz[i, pl.ds(0, 16)] = jnp.zeros((16,), jnp.float32)
            return 0

        lax.fori_loop(0, ZR, zrow, 0)

        def zcp(i, _):
            pltpu.sync_copy(zbuf, acc.at[pl.ds(s * RPS + i * ZR, ZR)])
            return 0

        lax.fori_loop(0, RPS // ZR, zcp, 0)

        @pl.when(c == 0)
        def _():
            def zcpz(i, _):
                pltpu.sync_copy(zbufz, accz.at[pl.ds(s * RPS + i * ZR, ZR)])
                return 0
            lax.fori_loop(0, RPS // ZR, zcpz, 0)

        plsc.subcore_barrier()
        base0 = s * per_s

        def chunk(i, _):
            base = base0 + i * CH
            pltpu.sync_copy(dst_hbm.at[pl.ds(base, CH)], idxb)
            pltpu.sync_copy(wv_hbm.at[pl.ds(base, CH), pl.ds(coff, 128)], buf)
            pltpu.sync_copy(buf, acc.at[idxb], add=True)

            @pl.when(c == 0)
            def _():
                pltpu.sync_copy(sexp_hbm.at[pl.ds(base, CH)], bufz)
                pltpu.sync_copy(bufz, accz.at[idxb], add=True)
            return 0

        lax.fori_loop(0, nfull, chunk, 0)
        tb = base0 + nfull * CH
        pltpu.sync_copy(dst_hbm.at[pl.ds(tb, tail)], tidxb)
        pltpu.sync_copy(wv_hbm.at[pl.ds(tb, tail), pl.ds(coff, 128)], tbuf)
        pltpu.sync_copy(tbuf, acc.at[tidxb], add=True)

        @pl.when(c == 0)
        def _():
            pltpu.sync_copy(sexp_hbm.at[pl.ds(tb, tail)], tbufz)
            pltpu.sync_copy(tbufz, accz.at[tidxb], add=True)

        plsc.subcore_barrier()
        pltpu.sync_copy(acc.at[pl.ds(s * RPS, RPS)],
                        out_hbm.at[pl.ds(s * RPS, RPS), pl.ds(coff, 128)])

        @pl.when(c == 0)
        def _():
            pltpu.sync_copy(accz.at[pl.ds(s * RPS, RPS)],
                            outz_hbm.at[pl.ds(s * RPS, RPS)])

    return kk(wv, sexp, dst)


def kernel(h, e, edge_index, Wq, Wk, Wv, We, Wo_h, bo_h, Wo_e, bo_e,
           W1h, b1h, W2h, b2h, W1e, b1e, W2e, b2e):
    src = edge_index[0].astype(jnp.int32)
    dst = edge_index[1].astype(jnp.int32)
    heads = jnp.arange(_D, dtype=jnp.int32) // _DH
    hsum = (heads[:, None] == jnp.arange(_H, dtype=jnp.int32)[None, :]
            ).astype(jnp.float32)
    hexp = hsum.T
    bo_h2 = bo_h.reshape(1, _D)
    bo_e2 = bo_e.reshape(1, _D)

    q, k, v = _qkv(h, Wq, Wk, Wv)
    ksrc, qdst, vsrc = _gather3(k, q, v, src, dst)
    eo, wv, sexp, stats_e = _edge1(e, ksrc, qdst, vsrc, We, Wo_e, bo_e2,
                                   hsum, hexp)
    wV, z = _scatter(wv, sexp, dst)
    ho, stats_h = _node1(wV, z, h, Wo_h, bo_h2, hexp)
    pre2_h, stats2_h = _ffn(ho, stats_h, _N, W1h, b1h.reshape(1, 2 * _D),
                            W2h, b2h.reshape(1, _D), 1000)
    pre2_e, stats2_e = _ffn(eo, stats_e, _E, W1e, b1e.reshape(1, 2 * _D),
                            W2e, b2e.reshape(1, _D), 1600)
    hh = _norm(pre2_h, stats2_h, _N, 1000)
    ee = _norm(pre2_e, stats2_e, _E, 1600)
    return (hh, ee)


# speedup vs baseline: 16.7593x; 16.7593x over previous
import functools

import jax
import jax.numpy as jnp
from jax import lax
from jax.experimental import pallas as pl
from jax.experimental.pallas import tpu as pltpu
from jax.experimental.pallas import tpu_sc as plsc

_N = 10000
_E = 160000
_D = 256
_H = 16
_DH = 16


def _qkv_body(h_ref, wq_ref, wk_ref, wv_ref, q_ref, k_ref, v_ref):
    hb = h_ref[...]
    q_ref[...] = jnp.dot(hb, wq_ref[...], preferred_element_type=jnp.float32)
    k_ref[...] = jnp.dot(hb, wk_ref[...], preferred_element_type=jnp.float32)
    v_ref[...] = jnp.dot(hb, wv_ref[...], preferred_element_type=jnp.float32)


def _qkv(h, Wq, Wk, Wv):
    BN = 1000
    mat = pl.BlockSpec((_D, _D), lambda i: (0, 0))
    blk = pl.BlockSpec((BN, _D), lambda i: (i, 0))
    return pl.pallas_call(
        _qkv_body,
        grid=(_N // BN,),
        in_specs=[blk, mat, mat, mat],
        out_specs=[blk, blk, blk],
        out_shape=[jax.ShapeDtypeStruct((_N, _D), jnp.float32)] * 3,
    )(h, Wq, Wk, Wv)


def _edge1_body(e_ref, ks_ref, qd_ref, vs_ref, we_ref, woe_ref, boe_ref,
                hsum_ref, hexp_ref, hrep_ref, eo_ref, wv_ref, sexp_ref,
                stats_ref):
    i = pl.program_id(0)
    eb = e_ref[...]
    pe = jnp.dot(eb, we_ref[...], preferred_element_type=jnp.float32)
    score = ks_ref[...] * qd_ref[...] * (pe * (1.0 / 4.0))
    hs = jnp.dot(score, hsum_ref[...], preferred_element_type=jnp.float32)
    sexp = jnp.exp(jnp.clip(hs, -5.0, 5.0))
    sexp_ref[...] = jnp.dot(sexp, hrep_ref[...],
                            preferred_element_type=jnp.float32)
    eo = eb + jnp.dot(score, woe_ref[...],
                      preferred_element_type=jnp.float32) + boe_ref[...]
    eo_ref[...] = eo
    wv_ref[...] = vs_ref[...] * jnp.dot(sexp, hexp_ref[...],
                                        preferred_element_type=jnp.float32)

    @pl.when(i == 0)
    def _():
        stats_ref[...] = jnp.zeros_like(stats_ref)

    stats_ref[0:1, :] += jnp.sum(eo, axis=0, keepdims=True)
    stats_ref[1:2, :] += jnp.sum(eo * eo, axis=0, keepdims=True)


def _edge1(e, ksrc, qdst, vsrc, We, Wo_e, bo_e, hsum, hexp, hrep):
    BE = 1600
    blk = pl.BlockSpec((BE, _D), lambda i: (i, 0))
    mat = pl.BlockSpec((_D, _D), lambda i: (0, 0))
    return pl.pallas_call(
        _edge1_body,
        grid=(_E // BE,),
        in_specs=[blk, blk, blk, blk, mat, mat,
                  pl.BlockSpec((1, _D), lambda i: (0, 0)),
                  pl.BlockSpec((_D, _H), lambda i: (0, 0)),
                  pl.BlockSpec((_H, _D), lambda i: (0, 0)),
                  pl.BlockSpec((_H, 128), lambda i: (0, 0))],
        out_specs=[blk, blk,
                   pl.BlockSpec((BE, 128), lambda i: (i, 0)),
                   pl.BlockSpec((8, _D), lambda i: (0, 0))],
        out_shape=[jax.ShapeDtypeStruct((_E, _D), jnp.float32),
                   jax.ShapeDtypeStruct((_E, _D), jnp.float32),
                   jax.ShapeDtypeStruct((_E, 128), jnp.float32),
                   jax.ShapeDtypeStruct((8, _D), jnp.float32)],
    )(e, ksrc, qdst, vsrc, We, Wo_e, bo_e, hsum, hexp, hrep)


def _node1_body(wv_ref, za_ref, zb_ref, h_ref, wo_ref, bo_ref, zsel_ref,
                ho_ref, stats_ref):
    i = pl.program_id(0)
    zrep = jnp.dot(za_ref[...] + zb_ref[...], zsel_ref[...],
                   preferred_element_type=jnp.float32)
    hattn = wv_ref[...] / (zrep + 1e-6)
    ho = h_ref[...] + jnp.dot(hattn, wo_ref[...],
                              preferred_element_type=jnp.float32) + bo_ref[...]
    ho_ref[...] = ho

    @pl.when(i == 0)
    def _():
        stats_ref[...] = jnp.zeros_like(stats_ref)

    stats_ref[0:1, :] += jnp.sum(ho, axis=0, keepdims=True)
    stats_ref[1:2, :] += jnp.sum(ho * ho, axis=0, keepdims=True)


def _node1(wV, za, zb, h, Wo_h, bo_h, zsel):
    BN = 1000
    blk = pl.BlockSpec((BN, _D), lambda i: (i, 0))
    zblk = pl.BlockSpec((BN, 128), lambda i: (i, 0))
    return pl.pallas_call(
        _node1_body,
        grid=(_N // BN,),
        in_specs=[blk, zblk, zblk, blk,
                  pl.BlockSpec((_D, _D), lambda i: (0, 0)),
                  pl.BlockSpec((1, _D), lambda i: (0, 0)),
                  pl.BlockSpec((128, _D), lambda i: (0, 0))],
        out_specs=[blk, pl.BlockSpec((8, _D), lambda i: (0, 0))],
        out_shape=[jax.ShapeDtypeStruct((_N, _D), jnp.float32),
                   jax.ShapeDtypeStruct((8, _D), jnp.float32)],
    )(wV, za, zb, h, Wo_h, bo_h, zsel)


def _ffn_body(count, x_ref, stats_ref, w1_ref, b1_ref, w2_ref, b2_ref,
              out_ref, stats2_ref):
    i = pl.program_id(0)
    mu = stats_ref[0:1, :] * (1.0 / count)
    var = stats_ref[1:2, :] * (1.0 / count) - mu * mu
    inv = lax.rsqrt(var + 1e-5)
    xb = (x_ref[...] - mu) * inv
    u = jnp.maximum(jnp.dot(xb, w1_ref[...],
                            preferred_element_type=jnp.float32) + b1_ref[...],
                    0.0)
    pre2 = xb + jnp.dot(u, w2_ref[...],
                        preferred_element_type=jnp.float32) + b2_ref[...]
    out_ref[...] = pre2

    @pl.when(i == 0)
    def _():
        stats2_ref[...] = jnp.zeros_like(stats2_ref)

    stats2_ref[0:1, :] += jnp.sum(pre2, axis=0, keepdims=True)
    stats2_ref[1:2, :] += jnp.sum(pre2 * pre2, axis=0, keepdims=True)


def _ffn(x, stats, count, W1, b1, W2, b2, bx):
    rows = x.shape[0]
    blk = pl.BlockSpec((bx, _D), lambda i: (i, 0))
    return pl.pallas_call(
        functools.partial(_ffn_body, float(count)),
        grid=(rows // bx,),
        in_specs=[blk, pl.BlockSpec((8, _D), lambda i: (0, 0)),
                  pl.BlockSpec((_D, 2 * _D), lambda i: (0, 0)),
                  pl.BlockSpec((1, 2 * _D), lambda i: (0, 0)),
                  pl.BlockSpec((2 * _D, _D), lambda i: (0, 0)),
                  pl.BlockSpec((1, _D), lambda i: (0, 0))],
        out_specs=[blk, pl.BlockSpec((8, _D), lambda i: (0, 0))],
        out_shape=[jax.ShapeDtypeStruct((rows, _D), jnp.float32),
                   jax.ShapeDtypeStruct((8, _D), jnp.float32)],
    )(x, stats, W1, b1, W2, b2)


def _norm_body(count, x_ref, stats_ref, out_ref):
    mu = stats_ref[0:1, :] * (1.0 / count)
    var = stats_ref[1:2, :] * (1.0 / count) - mu * mu
    inv = lax.rsqrt(var + 1e-5)
    out_ref[...] = (x_ref[...] - mu) * inv


def _norm(x, stats, count, bx):
    rows = x.shape[0]
    blk = pl.BlockSpec((bx, _D), lambda i: (i, 0))
    return pl.pallas_call(
        functools.partial(_norm_body, float(count)),
        grid=(rows // bx,),
        in_specs=[blk, pl.BlockSpec((8, _D), lambda i: (0, 0))],
        out_specs=blk,
        out_shape=jax.ShapeDtypeStruct((rows, _D), jnp.float32),
    )(x, stats)


def _gather3(ktab, qtab, vtab, src, dst):
    NW = 32
    per_w = _E // NW
    CH = 128
    nfull = per_w // CH
    tail = per_w - nfull * CH
    mesh = plsc.VectorSubcoreMesh(core_axis_name="c", subcore_axis_name="s",
                                  num_cores=2, num_subcores=16)

    @functools.partial(
        pl.kernel,
        out_type=[jax.ShapeDtypeStruct((_E, _D), jnp.float32)] * 3,
        mesh=mesh,
        scratch_types=[
            pltpu.VMEM((CH,), jnp.int32),
            pltpu.VMEM((CH,), jnp.int32),
            pltpu.VMEM((CH, _D), jnp.float32),
            pltpu.VMEM((tail,), jnp.int32),
            pltpu.VMEM((tail,), jnp.int32),
            pltpu.VMEM((tail, _D), jnp.float32),
            pltpu.SemaphoreType.DMA,
        ])
    def kk(k_hbm, q_hbm, v_hbm, src_hbm, dst_hbm, ks_hbm, qd_hbm, vs_hbm,
           idxs, idxd, rows, tidxs, tidxd, trows, sem):
        c = lax.axis_index("c")
        s = lax.axis_index("s")
        base0 = (s * 2 + c) * per_w

        def chunk(i, _):
            base = pl.multiple_of(base0 + i * CH, 8)
            pltpu.sync_copy(src_hbm.at[pl.ds(base, CH)], idxs)
            pltpu.sync_copy(dst_hbm.at[pl.ds(base, CH)], idxd)
            pltpu.async_copy(k_hbm.at[idxs], rows, sem).wait()
            pltpu.sync_copy(rows, ks_hbm.at[pl.ds(base, CH)])
            pltpu.async_copy(q_hbm.at[idxd], rows, sem).wait()
            pltpu.sync_copy(rows, qd_hbm.at[pl.ds(base, CH)])
            pltpu.async_copy(v_hbm.at[idxs], rows, sem).wait()
            pltpu.sync_copy(rows, vs_hbm.at[pl.ds(base, CH)])
            return 0

        lax.fori_loop(0, nfull, chunk, 0)
        base = base0 + nfull * CH
        pltpu.sync_copy(src_hbm.at[pl.ds(base, tail)], tidxs)
        pltpu.sync_copy(dst_hbm.at[pl.ds(base, tail)], tidxd)
        pltpu.async_copy(k_hbm.at[tidxs], trows, sem).wait()
        pltpu.sync_copy(trows, ks_hbm.at[pl.ds(base, tail)])
        pltpu.async_copy(q_hbm.at[tidxd], trows, sem).wait()
        pltpu.sync_copy(trows, qd_hbm.at[pl.ds(base, tail)])
        pltpu.async_copy(v_hbm.at[tidxs], trows, sem).wait()
        pltpu.sync_copy(trows, vs_hbm.at[pl.ds(base, tail)])

    return kk(ktab, qtab, vtab, src, dst)


_WIN = 640
_ZR = 40


def _win_start(s):
    return pl.multiple_of(
        jnp.minimum(s * (_N // 16) - lax.rem(s, 8), _N - _WIN), 8)


def _zero_rows(zbuf, acc, start):
    def zrow(i, _):
        def zlane(j, _):
            zbuf[i, pl.ds(j * 16, 16)] = jnp.zeros((16,), jnp.float32)
            return 0
        lax.fori_loop(0, 8, zlane, 0)
        return 0

    lax.fori_loop(0, _ZR, zrow, 0)

    def zcp(i, _):
        off = pl.multiple_of(start + i * _ZR, 8)
        pltpu.sync_copy(zbuf, acc.at[pl.ds(off, _ZR)])
        return 0

    lax.fori_loop(0, _WIN // _ZR, zcp, 0)


def _scatter_wv(wv, dst):
    per_s = _E // 16
    CH = 80
    nchunk = per_s // CH
    mesh = plsc.VectorSubcoreMesh(core_axis_name="c", subcore_axis_name="s",
                                  num_cores=2, num_subcores=16)

    @functools.partial(
        pl.kernel,
        out_type=jax.ShapeDtypeStruct((_N, _D), jnp.float32),
        mesh=mesh,
        scratch_types=[
            pltpu.VMEM_SHARED((_N, 128), jnp.float32),
            pltpu.VMEM((_ZR, 128), jnp.float32),
            pltpu.VMEM((CH, 128), jnp.float32),
            pltpu.VMEM((CH,), jnp.int32),
        ])
    def kk(wv_hbm, dst_hbm, out_hbm, acc, zbuf, buf, idxb):
        c = lax.axis_index("c")
        s = lax.axis_index("s")
        coff = c * 128
        start = _win_start(s)
        _zero_rows(zbuf, acc, start)
        plsc.subcore_barrier()
        base0 = s * per_s

        def chunk(i, _):
            base = pl.multiple_of(base0 + i * CH, 8)
            pltpu.sync_copy(dst_hbm.at[pl.ds(base, CH)], idxb)
            pltpu.sync_copy(wv_hbm.at[pl.ds(base, CH), pl.ds(coff, 128)], buf)
            pltpu.sync_copy(buf, acc.at[idxb], add=True)
            return 0

        lax.fori_loop(0, nchunk, chunk, 0)
        plsc.subcore_barrier()
        pltpu.sync_copy(acc.at[pl.ds(start, _WIN)],
                        out_hbm.at[pl.ds(start, _WIN), pl.ds(coff, 128)])

    return kk(wv, dst)


def _scatter_z(sexp128, dst):
    half = _E // 2
    per_s = half // 16
    CH = 40
    nchunk = per_s // CH
    mesh = plsc.VectorSubcoreMesh(core_axis_name="c", subcore_axis_name="s",
                                  num_cores=2, num_subcores=16)

    @functools.partial(
        pl.kernel,
        out_type=[jax.ShapeDtypeStruct((_N, 128), jnp.float32),
                  jax.ShapeDtypeStruct((_N, 128), jnp.float32)],
        mesh=mesh,
        scratch_types=[
            pltpu.VMEM_SHARED((_N, 128), jnp.float32),
            pltpu.VMEM((_ZR, 128), jnp.float32),
            pltpu.VMEM((CH, 128), jnp.float32),
            pltpu.VMEM((CH,), jnp.int32),
        ])
    def kk(se_hbm, dst_hbm, outa_hbm, outb_hbm, acc, zbuf, buf, idxb):
        c = lax.axis_index("c")
        s = lax.axis_index("s")
        start = _win_start(s)
        _zero_rows(zbuf, acc, start)
        plsc.subcore_barrier()
        base0 = c * half + s * per_s

        def chunk(i, _):
            base = pl.multiple_of(base0 + i * CH, 8)
            pltpu.sync_copy(dst_hbm.at[pl.ds(base, CH)], idxb)
            pltpu.sync_copy(se_hbm.at[pl.ds(base, CH)], buf)
            pltpu.sync_copy(buf, acc.at[idxb], add=True)
            return 0

        lax.fori_loop(0, nchunk, chunk, 0)
        plsc.subcore_barrier()

        @pl.when(c == 0)
        def _():
            pltpu.sync_copy(acc.at[pl.ds(start, _WIN)],
                            outa_hbm.at[pl.ds(start, _WIN)])

        @pl.when(c == 1)
        def _():
            pltpu.sync_copy(acc.at[pl.ds(start, _WIN)],
                            outb_hbm.at[pl.ds(start, _WIN)])

    return kk(sexp128, dst)


def kernel(h, e, edge_index, Wq, Wk, Wv, We, Wo_h, bo_h, Wo_e, bo_e,
           W1h, b1h, W2h, b2h, W1e, b1e, W2e, b2e):
    src = edge_index[0].astype(jnp.int32)
    dst = edge_index[1].astype(jnp.int32)
    heads = jnp.arange(_D, dtype=jnp.int32) // _DH
    hsum = (heads[:, None] == jnp.arange(_H, dtype=jnp.int32)[None, :]
            ).astype(jnp.float32)
    hexp = hsum.T
    j8 = jnp.arange(128, dtype=jnp.int32) // 8
    hrep = (jnp.arange(_H, dtype=jnp.int32)[:, None] == j8[None, :]
            ).astype(jnp.float32)
    zsel = (j8[:, None] == heads[None, :]).astype(jnp.float32) * (1.0 / 8.0)
    bo_h2 = bo_h.reshape(1, _D)
    bo_e2 = bo_e.reshape(1, _D)

    q, k, v = _qkv(h, Wq, Wk, Wv)
    ksrc, qdst, vsrc = _gather3(k, q, v, src, dst)
    eo, wv, sexp128, stats_e = _edge1(e, ksrc, qdst, vsrc, We, Wo_e, bo_e2,
                                      hsum, hexp, hrep)
    wV = _scatter_wv(wv, dst)
    za, zb = _scatter_z(sexp128, dst)
    ho, stats_h = _node1(wV, za, zb, h, Wo_h, bo_h2, zsel)
    pre2_h, stats2_h = _ffn(ho, stats_h, _N, W1h, b1h.reshape(1, 2 * _D),
                            W2h, b2h.reshape(1, _D), 1000)
    pre2_e, stats2_e = _ffn(eo, stats_e, _E, W1e, b1e.reshape(1, 2 * _D),
                            W2e, b2e.reshape(1, _D), 1600)
    hh = _norm(pre2_h, stats2_h, _N, 1000)
    ee = _norm(pre2_e, stats2_e, _E, 1600)
    return (hh, ee)


# speedup vs baseline: 18.2221x; 1.0873x over previous
import functools

import jax
import jax.numpy as jnp
from jax import lax
from jax.experimental import pallas as pl
from jax.experimental.pallas import tpu as pltpu
from jax.experimental.pallas import tpu_sc as plsc

_N = 10000
_E = 160000
_D = 256
_H = 16
_DH = 16


def _qkv_body(h_ref, wq_ref, wk_ref, wv_ref, q_ref, k_ref, v_ref):
    hb = h_ref[...]
    q_ref[...] = jnp.dot(hb, wq_ref[...], preferred_element_type=jnp.float32)
    k_ref[...] = jnp.dot(hb, wk_ref[...], preferred_element_type=jnp.float32)
    v_ref[...] = jnp.dot(hb, wv_ref[...], preferred_element_type=jnp.float32)


def _qkv(h, Wq, Wk, Wv):
    BN = 1000
    mat = pl.BlockSpec((_D, _D), lambda i: (0, 0))
    blk = pl.BlockSpec((BN, _D), lambda i: (i, 0))
    return pl.pallas_call(
        _qkv_body,
        grid=(_N // BN,),
        in_specs=[blk, mat, mat, mat],
        out_specs=[blk, blk, blk],
        out_shape=[jax.ShapeDtypeStruct((_N, _D), jnp.float32)] * 3,
    )(h, Wq, Wk, Wv)


def _edge1_body(e_ref, ks_ref, qd_ref, vs_ref, we_ref, woe_ref, boe_ref,
                hsum_ref, hexp_ref, hrep_ref, eo_ref, wv_ref, sexp_ref,
                stats_ref):
    i = pl.program_id(0)
    eb = e_ref[...]
    pe = jnp.dot(eb, we_ref[...], preferred_element_type=jnp.float32)
    score = ks_ref[...] * qd_ref[...] * (pe * (1.0 / 4.0))
    hs = jnp.dot(score, hsum_ref[...], preferred_element_type=jnp.float32)
    sexp = jnp.exp(jnp.clip(hs, -5.0, 5.0))
    sexp_ref[...] = jnp.dot(sexp, hrep_ref[...],
                            preferred_element_type=jnp.float32)
    eo = eb + jnp.dot(score, woe_ref[...],
                      preferred_element_type=jnp.float32) + boe_ref[...]
    eo_ref[...] = eo
    wv_ref[...] = vs_ref[...] * jnp.dot(sexp, hexp_ref[...],
                                        preferred_element_type=jnp.float32)

    @pl.when(i == 0)
    def _():
        stats_ref[...] = jnp.zeros_like(stats_ref)

    stats_ref[0:1, :] += jnp.sum(eo, axis=0, keepdims=True)
    stats_ref[1:2, :] += jnp.sum(eo * eo, axis=0, keepdims=True)


def _edge1(e, ksrc, qdst, vsrc, We, Wo_e, bo_e, hsum, hexp, hrep):
    BE = 1600
    blk = pl.BlockSpec((BE, _D), lambda i: (i, 0))
    mat = pl.BlockSpec((_D, _D), lambda i: (0, 0))
    return pl.pallas_call(
        _edge1_body,
        grid=(_E // BE,),
        in_specs=[blk, blk, blk, blk, mat, mat,
                  pl.BlockSpec((1, _D), lambda i: (0, 0)),
                  pl.BlockSpec((_D, _H), lambda i: (0, 0)),
                  pl.BlockSpec((_H, _D), lambda i: (0, 0)),
                  pl.BlockSpec((_H, 128), lambda i: (0, 0))],
        out_specs=[blk, blk,
                   pl.BlockSpec((BE, 128), lambda i: (i, 0)),
                   pl.BlockSpec((8, _D), lambda i: (0, 0))],
        out_shape=[jax.ShapeDtypeStruct((_E, _D), jnp.float32),
                   jax.ShapeDtypeStruct((_E, _D), jnp.float32),
                   jax.ShapeDtypeStruct((_E, 128), jnp.float32),
                   jax.ShapeDtypeStruct((8, _D), jnp.float32)],
    )(e, ksrc, qdst, vsrc, We, Wo_e, bo_e, hsum, hexp, hrep)


def _node1_body(wv_ref, za_ref, zb_ref, h_ref, wo_ref, bo_ref, zsel_ref,
                ho_ref, stats_ref):
    i = pl.program_id(0)
    zrep = jnp.dot(za_ref[...] + zb_ref[...], zsel_ref[...],
                   preferred_element_type=jnp.float32)
    hattn = wv_ref[...] / (zrep + 1e-6)
    ho = h_ref[...] + jnp.dot(hattn, wo_ref[...],
                              preferred_element_type=jnp.float32) + bo_ref[...]
    ho_ref[...] = ho

    @pl.when(i == 0)
    def _():
        stats_ref[...] = jnp.zeros_like(stats_ref)

    stats_ref[0:1, :] += jnp.sum(ho, axis=0, keepdims=True)
    stats_ref[1:2, :] += jnp.sum(ho * ho, axis=0, keepdims=True)


def _node1(wV, za, zb, h, Wo_h, bo_h, zsel):
    BN = 1000
    blk = pl.BlockSpec((BN, _D), lambda i: (i, 0))
    zblk = pl.BlockSpec((BN, 128), lambda i: (i, 0))
    return pl.pallas_call(
        _node1_body,
        grid=(_N // BN,),
        in_specs=[blk, zblk, zblk, blk,
                  pl.BlockSpec((_D, _D), lambda i: (0, 0)),
                  pl.BlockSpec((1, _D), lambda i: (0, 0)),
                  pl.BlockSpec((128, _D), lambda i: (0, 0))],
        out_specs=[blk, pl.BlockSpec((8, _D), lambda i: (0, 0))],
        out_shape=[jax.ShapeDtypeStruct((_N, _D), jnp.float32),
                   jax.ShapeDtypeStruct((8, _D), jnp.float32)],
    )(wV, za, zb, h, Wo_h, bo_h, zsel)


def _ffn_body(count, x_ref, stats_ref, w1_ref, b1_ref, w2_ref, b2_ref,
              out_ref, stats2_ref):
    i = pl.program_id(0)
    mu = stats_ref[0:1, :] * (1.0 / count)
    var = stats_ref[1:2, :] * (1.0 / count) - mu * mu
    inv = lax.rsqrt(var + 1e-5)
    xb = (x_ref[...] - mu) * inv
    u = jnp.maximum(jnp.dot(xb, w1_ref[...],
                            preferred_element_type=jnp.float32) + b1_ref[...],
                    0.0)
    pre2 = xb + jnp.dot(u, w2_ref[...],
                        preferred_element_type=jnp.float32) + b2_ref[...]
    out_ref[...] = pre2

    @pl.when(i == 0)
    def _():
        stats2_ref[...] = jnp.zeros_like(stats2_ref)

    stats2_ref[0:1, :] += jnp.sum(pre2, axis=0, keepdims=True)
    stats2_ref[1:2, :] += jnp.sum(pre2 * pre2, axis=0, keepdims=True)


def _ffn(x, stats, count, W1, b1, W2, b2, bx):
    rows = x.shape[0]
    blk = pl.BlockSpec((bx, _D), lambda i: (i, 0))
    return pl.pallas_call(
        functools.partial(_ffn_body, float(count)),
        grid=(rows // bx,),
        in_specs=[blk, pl.BlockSpec((8, _D), lambda i: (0, 0)),
                  pl.BlockSpec((_D, 2 * _D), lambda i: (0, 0)),
                  pl.BlockSpec((1, 2 * _D), lambda i: (0, 0)),
                  pl.BlockSpec((2 * _D, _D), lambda i: (0, 0)),
                  pl.BlockSpec((1, _D), lambda i: (0, 0))],
        out_specs=[blk, pl.BlockSpec((8, _D), lambda i: (0, 0))],
        out_shape=[jax.ShapeDtypeStruct((rows, _D), jnp.float32),
                   jax.ShapeDtypeStruct((8, _D), jnp.float32)],
    )(x, stats, W1, b1, W2, b2)


def _norm_body(count, x_ref, stats_ref, out_ref):
    mu = stats_ref[0:1, :] * (1.0 / count)
    var = stats_ref[1:2, :] * (1.0 / count) - mu * mu
    inv = lax.rsqrt(var + 1e-5)
    out_ref[...] = (x_ref[...] - mu) * inv


def _norm(x, stats, count, bx):
    rows = x.shape[0]
    blk = pl.BlockSpec((bx, _D), lambda i: (i, 0))
    return pl.pallas_call(
        functools.partial(_norm_body, float(count)),
        grid=(rows // bx,),
        in_specs=[blk, pl.BlockSpec((8, _D), lambda i: (0, 0))],
        out_specs=blk,
        out_shape=jax.ShapeDtypeStruct((rows, _D), jnp.float32),
    )(x, stats)


def _gather3(ktab, qtab, vtab, src, dst):
    NW = 32
    per_w = _E // NW
    CH = 64
    nfull = per_w // CH
    tail = per_w - nfull * CH
    mesh = plsc.VectorSubcoreMesh(core_axis_name="c", subcore_axis_name="s",
                                  num_cores=2, num_subcores=16)

    @functools.partial(
        pl.kernel,
        out_type=[jax.ShapeDtypeStruct((_E, _D), jnp.float32)] * 3,
        mesh=mesh,
        scratch_types=[
            [pltpu.VMEM((CH,), jnp.int32)] * 2,
            [pltpu.VMEM((CH,), jnp.int32)] * 2,
            [pltpu.VMEM((CH, _D), jnp.float32)] * 2,
            [pltpu.VMEM((CH, _D), jnp.float32)] * 2,
            [pltpu.VMEM((CH, _D), jnp.float32)] * 2,
            [pltpu.SemaphoreType.DMA] * 6,
            [pltpu.SemaphoreType.DMA] * 6,
        ])
    def kk(k_hbm, q_hbm, v_hbm, src_hbm, dst_hbm, ks_hbm, qd_hbm, vs_hbm,
           idxs, idxd, krows, qrows, vrows, gsem, wsem):
        c = lax.axis_index("c")
        s = lax.axis_index("s")
        base0 = (s * 2 + c) * per_w

        def dstep(g, _):
            for b in (0, 1):
                i = 2 * g + b
                base = pl.multiple_of(base0 + i * CH, 8)

                @pl.when(g > 0)
                def _():
                    pltpu.make_async_copy(
                        krows[b], ks_hbm.at[pl.ds(base, CH)],
                        wsem[3 * b]).wait()
                    pltpu.make_async_copy(
                        qrows[b], qd_hbm.at[pl.ds(base, CH)],
                        wsem[3 * b + 1]).wait()
                    pltpu.make_async_copy(
                        vrows[b], vs_hbm.at[pl.ds(base, CH)],
                        wsem[3 * b + 2]).wait()

                pltpu.sync_copy(src_hbm.at[pl.ds(base, CH)], idxs[b])
                pltpu.sync_copy(dst_hbm.at[pl.ds(base, CH)], idxd[b])
                pltpu.async_copy(k_hbm.at[idxs[b]], krows[b], gsem[3 * b])
                pltpu.async_copy(q_hbm.at[idxd[b]], qrows[b],
                                 gsem[3 * b + 1])
                pltpu.async_copy(v_hbm.at[idxs[b]], vrows[b],
                                 gsem[3 * b + 2])
            for b in (0, 1):
                i = 2 * g + b
                base = pl.multiple_of(base0 + i * CH, 8)
                pltpu.make_async_copy(k_hbm.at[idxs[b]], krows[b],
                                      gsem[3 * b]).wait()
                pltpu.async_copy(krows[b], ks_hbm.at[pl.ds(base, CH)],
                                 wsem[3 * b])
                pltpu.make_async_copy(q_hbm.at[idxd[b]], qrows[b],
                                      gsem[3 * b + 1]).wait()
                pltpu.async_copy(qrows[b], qd_hbm.at[pl.ds(base, CH)],
                                 wsem[3 * b + 1])
                pltpu.make_async_copy(v_hbm.at[idxs[b]], vrows[b],
                                      gsem[3 * b + 2]).wait()
                pltpu.async_copy(vrows[b], vs_hbm.at[pl.ds(base, CH)],
                                 wsem[3 * b + 2])
            return 0

        lax.fori_loop(0, nfull // 2, dstep, 0)
        for b in (0, 1):
            base = pl.multiple_of(base0, 8)
            pltpu.make_async_copy(krows[b], ks_hbm.at[pl.ds(base, CH)],
                                  wsem[3 * b]).wait()
            pltpu.make_async_copy(qrows[b], qd_hbm.at[pl.ds(base, CH)],
                                  wsem[3 * b + 1]).wait()
            pltpu.make_async_copy(vrows[b], vs_hbm.at[pl.ds(base, CH)],
                                  wsem[3 * b + 2]).wait()
        tb = pl.multiple_of(base0 + nfull * CH, 8)
        pltpu.sync_copy(src_hbm.at[pl.ds(tb, tail)],
                        idxs[0].at[pl.ds(0, tail)])
        pltpu.sync_copy(dst_hbm.at[pl.ds(tb, tail)],
                        idxd[0].at[pl.ds(0, tail)])
        pltpu.async_copy(k_hbm.at[idxs[0].at[pl.ds(0, tail)]],
                         krows[0].at[pl.ds(0, tail)], gsem[0]).wait()
        pltpu.sync_copy(krows[0].at[pl.ds(0, tail)],
                        ks_hbm.at[pl.ds(tb, tail)])
        pltpu.async_copy(q_hbm.at[idxd[0].at[pl.ds(0, tail)]],
                         qrows[0].at[pl.ds(0, tail)], gsem[1]).wait()
        pltpu.sync_copy(qrows[0].at[pl.ds(0, tail)],
                        qd_hbm.at[pl.ds(tb, tail)])
        pltpu.async_copy(v_hbm.at[idxs[0].at[pl.ds(0, tail)]],
                         vrows[0].at[pl.ds(0, tail)], gsem[2]).wait()
        pltpu.sync_copy(vrows[0].at[pl.ds(0, tail)],
                        vs_hbm.at[pl.ds(tb, tail)])

    return kk(ktab, qtab, vtab, src, dst)


_WIN = 640
_ZR = 40


def _win_start(s):
    return pl.multiple_of(
        jnp.minimum(s * (_N // 16) - lax.rem(s, 8), _N - _WIN), 8)


def _zero_rows(zbuf, acc, start):
    def zrow(i, _):
        def zlane(j, _):
            zbuf[i, pl.ds(j * 16, 16)] = jnp.zeros((16,), jnp.float32)
            return 0
        lax.fori_loop(0, 8, zlane, 0)
        return 0

    lax.fori_loop(0, _ZR, zrow, 0)

    def zcp(i, _):
        off = pl.multiple_of(start + i * _ZR, 8)
        pltpu.sync_copy(zbuf, acc.at[pl.ds(off, _ZR)])
        return 0

    lax.fori_loop(0, _WIN // _ZR, zcp, 0)


def _scatter_wv(wv, dst):
    per_s = _E // 16
    CH = 80
    nchunk = per_s // CH
    mesh = plsc.VectorSubcoreMesh(core_axis_name="c", subcore_axis_name="s",
                                  num_cores=2, num_subcores=16)

    @functools.partial(
        pl.kernel,
        out_type=jax.ShapeDtypeStruct((_N, _D), jnp.float32),
        mesh=mesh,
        scratch_types=[
            pltpu.VMEM_SHARED((_N, 128), jnp.float32),
            pltpu.VMEM((_ZR, 128), jnp.float32),
            pltpu.VMEM((CH, 128), jnp.float32),
            pltpu.VMEM((CH,), jnp.int32),
        ])
    def kk(wv_hbm, dst_hbm, out_hbm, acc, zbuf, buf, idxb):
        c = lax.axis_index("c")
        s = lax.axis_index("s")
        coff = c * 128
        start = _win_start(s)
        _zero_rows(zbuf, acc, start)
        plsc.subcore_barrier()
        base0 = s * per_s

        def chunk(i, _):
            base = pl.multiple_of(base0 + i * CH, 8)
            pltpu.sync_copy(dst_hbm.at[pl.ds(base, CH)], idxb)
            pltpu.sync_copy(wv_hbm.at[pl.ds(base, CH), pl.ds(coff, 128)], buf)
            pltpu.sync_copy(buf, acc.at[idxb], add=True)
            return 0

        lax.fori_loop(0, nchunk, chunk, 0)
        plsc.subcore_barrier()
        pltpu.sync_copy(acc.at[pl.ds(start, _WIN)],
                        out_hbm.at[pl.ds(start, _WIN), pl.ds(coff, 128)])

    return kk(wv, dst)


def _scatter_z(sexp128, dst):
    half = _E // 2
    per_s = half // 16
    CH = 40
    nchunk = per_s // CH
    mesh = plsc.VectorSubcoreMesh(core_axis_name="c", subcore_axis_name="s",
                                  num_cores=2, num_subcores=16)

    @functools.partial(
        pl.kernel,
        out_type=[jax.ShapeDtypeStruct((_N, 128), jnp.float32),
                  jax.ShapeDtypeStruct((_N, 128), jnp.float32)],
        mesh=mesh,
        scratch_types=[
            pltpu.VMEM_SHARED((_N, 128), jnp.float32),
            pltpu.VMEM((_ZR, 128), jnp.float32),
            pltpu.VMEM((CH, 128), jnp.float32),
            pltpu.VMEM((CH,), jnp.int32),
        ])
    def kk(se_hbm, dst_hbm, outa_hbm, outb_hbm, acc, zbuf, buf, idxb):
        c = lax.axis_index("c")
        s = lax.axis_index("s")
        start = _win_start(s)
        _zero_rows(zbuf, acc, start)
        plsc.subcore_barrier()
        base0 = c * half + s * per_s

        def chunk(i, _):
            base = pl.multiple_of(base0 + i * CH, 8)
            pltpu.sync_copy(dst_hbm.at[pl.ds(base, CH)], idxb)
            pltpu.sync_copy(se_hbm.at[pl.ds(base, CH)], buf)
            pltpu.sync_copy(buf, acc.at[idxb], add=True)
            return 0

        lax.fori_loop(0, nchunk, chunk, 0)
        plsc.subcore_barrier()

        @pl.when(c == 0)
        def _():
            pltpu.sync_copy(acc.at[pl.ds(start, _WIN)],
                            outa_hbm.at[pl.ds(start, _WIN)])

        @pl.when(c == 1)
        def _():
            pltpu.sync_copy(acc.at[pl.ds(start, _WIN)],
                            outb_hbm.at[pl.ds(start, _WIN)])

    return kk(sexp128, dst)


def kernel(h, e, edge_index, Wq, Wk, Wv, We, Wo_h, bo_h, Wo_e, bo_e,
           W1h, b1h, W2h, b2h, W1e, b1e, W2e, b2e):
    src = edge_index[0].astype(jnp.int32)
    dst = edge_index[1].astype(jnp.int32)
    heads = jnp.arange(_D, dtype=jnp.int32) // _DH
    hsum = (heads[:, None] == jnp.arange(_H, dtype=jnp.int32)[None, :]
            ).astype(jnp.float32)
    hexp = hsum.T
    j8 = jnp.arange(128, dtype=jnp.int32) // 8
    hrep = (jnp.arange(_H, dtype=jnp.int32)[:, None] == j8[None, :]
            ).astype(jnp.float32)
    zsel = (j8[:, None] == heads[None, :]).astype(jnp.float32) * (1.0 / 8.0)
    bo_h2 = bo_h.reshape(1, _D)
    bo_e2 = bo_e.reshape(1, _D)

    q, k, v = _qkv(h, Wq, Wk, Wv)
    ksrc, qdst, vsrc = _gather3(k, q, v, src, dst)
    eo, wv, sexp128, stats_e = _edge1(e, ksrc, qdst, vsrc, We, Wo_e, bo_e2,
                                      hsum, hexp, hrep)
    wV = _scatter_wv(wv, dst)
    za, zb = _scatter_z(sexp128, dst)
    ho, stats_h = _node1(wV, za, zb, h, Wo_h, bo_h2, zsel)
    pre2_h, stats2_h = _ffn(ho, stats_h, _N, W1h, b1h.reshape(1, 2 * _D),
                            W2h, b2h.reshape(1, _D), 1000)
    pre2_e, stats2_e = _ffn(eo, stats_e, _E, W1e, b1e.reshape(1, 2 * _D),
                            W2e, b2e.reshape(1, _D), 1600)
    hh = _norm(pre2_h, stats2_h, _N, 1000)
    ee = _norm(pre2_e, stats2_e, _E, 1600)
    return (hh, ee)


# speedup vs baseline: 21.3030x; 1.1691x over previous
import functools

import jax
import jax.numpy as jnp
from jax import lax
from jax.experimental import pallas as pl
from jax.experimental.pallas import tpu as pltpu
from jax.experimental.pallas import tpu_sc as plsc

_N = 10000
_E = 160000
_D = 256
_H = 16
_DH = 16


def _qkv_body(h_ref, wq_ref, wk_ref, wv_ref, q_ref, k_ref, v_ref):
    hb = h_ref[...]
    q_ref[...] = jnp.dot(hb, wq_ref[...], preferred_element_type=jnp.float32)
    k_ref[...] = jnp.dot(hb, wk_ref[...], preferred_element_type=jnp.float32)
    v_ref[...] = jnp.dot(hb, wv_ref[...], preferred_element_type=jnp.float32)


def _qkv(h, Wq, Wk, Wv):
    BN = 1000
    mat = pl.BlockSpec((_D, _D), lambda i: (0, 0))
    blk = pl.BlockSpec((BN, _D), lambda i: (i, 0))
    return pl.pallas_call(
        _qkv_body,
        grid=(_N // BN,),
        in_specs=[blk, mat, mat, mat],
        out_specs=[blk, blk, blk],
        out_shape=[jax.ShapeDtypeStruct((_N, _D), jnp.float32)] * 3,
    )(h, Wq, Wk, Wv)


def _edge1_body(e_ref, ks_ref, qd_ref, vs_ref, we_ref, woe_ref, boe_ref,
                hsum_ref, hexp_ref, hrep_ref, eo_ref, wv_ref, sexp_ref,
                stats_ref):
    i = pl.program_id(0)
    eb = e_ref[...]
    pe = jnp.dot(eb, we_ref[...], preferred_element_type=jnp.float32)
    score = ks_ref[...] * qd_ref[...] * (pe * (1.0 / 4.0))
    hs = jnp.dot(score, hsum_ref[...], preferred_element_type=jnp.float32)
    sexp = jnp.exp(jnp.clip(hs, -5.0, 5.0))
    sexp_ref[...] = jnp.dot(sexp, hrep_ref[...],
                            preferred_element_type=jnp.float32)
    eo = eb + jnp.dot(score, woe_ref[...],
                      preferred_element_type=jnp.float32) + boe_ref[...]
    eo_ref[...] = eo
    wv_ref[...] = vs_ref[...] * jnp.dot(sexp, hexp_ref[...],
                                        preferred_element_type=jnp.float32)

    @pl.when(i == 0)
    def _():
        stats_ref[...] = jnp.zeros_like(stats_ref)

    stats_ref[0:1, :] += jnp.sum(eo, axis=0, keepdims=True)
    stats_ref[1:2, :] += jnp.sum(eo * eo, axis=0, keepdims=True)


def _edge1(e, ksrc, qdst, vsrc, We, Wo_e, bo_e, hsum, hexp, hrep):
    BE = 1600
    blk = pl.BlockSpec((BE, _D), lambda i: (i, 0))
    mat = pl.BlockSpec((_D, _D), lambda i: (0, 0))
    return pl.pallas_call(
        _edge1_body,
        grid=(_E // BE,),
        in_specs=[blk, blk, blk, blk, mat, mat,
                  pl.BlockSpec((1, _D), lambda i: (0, 0)),
                  pl.BlockSpec((_D, _H), lambda i: (0, 0)),
                  pl.BlockSpec((_H, _D), lambda i: (0, 0)),
                  pl.BlockSpec((_H, 128), lambda i: (0, 0))],
        out_specs=[blk, blk,
                   pl.BlockSpec((BE, 128), lambda i: (i, 0)),
                   pl.BlockSpec((8, _D), lambda i: (0, 0))],
        out_shape=[jax.ShapeDtypeStruct((_E, _D), jnp.float32),
                   jax.ShapeDtypeStruct((_E, _D), jnp.float32),
                   jax.ShapeDtypeStruct((_E, 128), jnp.float32),
                   jax.ShapeDtypeStruct((8, _D), jnp.float32)],
    )(e, ksrc, qdst, vsrc, We, Wo_e, bo_e, hsum, hexp, hrep)


def _node1_body(wv_ref, za_ref, zb_ref, h_ref, wo_ref, bo_ref, zsel_ref,
                ho_ref, stats_ref):
    i = pl.program_id(0)
    zrep = jnp.dot(za_ref[...] + zb_ref[...], zsel_ref[...],
                   preferred_element_type=jnp.float32)
    hattn = wv_ref[...] / (zrep + 1e-6)
    ho = h_ref[...] + jnp.dot(hattn, wo_ref[...],
                              preferred_element_type=jnp.float32) + bo_ref[...]
    ho_ref[...] = ho

    @pl.when(i == 0)
    def _():
        stats_ref[...] = jnp.zeros_like(stats_ref)

    stats_ref[0:1, :] += jnp.sum(ho, axis=0, keepdims=True)
    stats_ref[1:2, :] += jnp.sum(ho * ho, axis=0, keepdims=True)


def _node1(wV, za, zb, h, Wo_h, bo_h, zsel):
    BN = 1000
    blk = pl.BlockSpec((BN, _D), lambda i: (i, 0))
    zblk = pl.BlockSpec((BN, 128), lambda i: (i, 0))
    return pl.pallas_call(
        _node1_body,
        grid=(_N // BN,),
        in_specs=[blk, zblk, zblk, blk,
                  pl.BlockSpec((_D, _D), lambda i: (0, 0)),
                  pl.BlockSpec((1, _D), lambda i: (0, 0)),
                  pl.BlockSpec((128, _D), lambda i: (0, 0))],
        out_specs=[blk, pl.BlockSpec((8, _D), lambda i: (0, 0))],
        out_shape=[jax.ShapeDtypeStruct((_N, _D), jnp.float32),
                   jax.ShapeDtypeStruct((8, _D), jnp.float32)],
    )(wV, za, zb, h, Wo_h, bo_h, zsel)


def _ffn_body(count, x_ref, stats_ref, w1_ref, b1_ref, w2_ref, b2_ref,
              out_ref, stats2_ref):
    i = pl.program_id(0)
    mu = stats_ref[0:1, :] * (1.0 / count)
    var = stats_ref[1:2, :] * (1.0 / count) - mu * mu
    inv = lax.rsqrt(var + 1e-5)
    xb = (x_ref[...] - mu) * inv
    u = jnp.maximum(jnp.dot(xb, w1_ref[...],
                            preferred_element_type=jnp.float32) + b1_ref[...],
                    0.0)
    pre2 = xb + jnp.dot(u, w2_ref[...],
                        preferred_element_type=jnp.float32) + b2_ref[...]
    out_ref[...] = pre2

    @pl.when(i == 0)
    def _():
        stats2_ref[...] = jnp.zeros_like(stats2_ref)

    stats2_ref[0:1, :] += jnp.sum(pre2, axis=0, keepdims=True)
    stats2_ref[1:2, :] += jnp.sum(pre2 * pre2, axis=0, keepdims=True)


def _ffn(x, stats, count, W1, b1, W2, b2, bx):
    rows = x.shape[0]
    blk = pl.BlockSpec((bx, _D), lambda i: (i, 0))
    return pl.pallas_call(
        functools.partial(_ffn_body, float(count)),
        grid=(rows // bx,),
        in_specs=[blk, pl.BlockSpec((8, _D), lambda i: (0, 0)),
                  pl.BlockSpec((_D, 2 * _D), lambda i: (0, 0)),
                  pl.BlockSpec((1, 2 * _D), lambda i: (0, 0)),
                  pl.BlockSpec((2 * _D, _D), lambda i: (0, 0)),
                  pl.BlockSpec((1, _D), lambda i: (0, 0))],
        out_specs=[blk, pl.BlockSpec((8, _D), lambda i: (0, 0))],
        out_shape=[jax.ShapeDtypeStruct((rows, _D), jnp.float32),
                   jax.ShapeDtypeStruct((8, _D), jnp.float32)],
    )(x, stats, W1, b1, W2, b2)


def _norm_body(count, x_ref, stats_ref, out_ref):
    mu = stats_ref[0:1, :] * (1.0 / count)
    var = stats_ref[1:2, :] * (1.0 / count) - mu * mu
    inv = lax.rsqrt(var + 1e-5)
    out_ref[...] = (x_ref[...] - mu) * inv


def _norm(x, stats, count, bx):
    rows = x.shape[0]
    blk = pl.BlockSpec((bx, _D), lambda i: (i, 0))
    return pl.pallas_call(
        functools.partial(_norm_body, float(count)),
        grid=(rows // bx,),
        in_specs=[blk, pl.BlockSpec((8, _D), lambda i: (0, 0))],
        out_specs=blk,
        out_shape=jax.ShapeDtypeStruct((rows, _D), jnp.float32),
    )(x, stats)


def _gather3(ktab, qtab, vtab, src, dst):
    NW = 32
    per_w = _E // NW
    CH = 64
    nfull = per_w // CH
    tail = per_w - nfull * CH
    mesh = plsc.VectorSubcoreMesh(core_axis_name="c", subcore_axis_name="s",
                                  num_cores=2, num_subcores=16)

    @functools.partial(
        pl.kernel,
        out_type=[jax.ShapeDtypeStruct((_E, _D), jnp.float32)] * 3,
        mesh=mesh,
        scratch_types=[
            [pltpu.VMEM((CH,), jnp.int32)] * 2,
            [pltpu.VMEM((CH,), jnp.int32)] * 2,
            [pltpu.VMEM((CH, _D), jnp.float32)] * 2,
            [pltpu.VMEM((CH, _D), jnp.float32)] * 2,
            [pltpu.VMEM((CH, _D), jnp.float32)] * 2,
            [pltpu.SemaphoreType.DMA] * 6,
            [pltpu.SemaphoreType.DMA] * 6,
        ])
    def kk(k_hbm, q_hbm, v_hbm, src_hbm, dst_hbm, ks_hbm, qd_hbm, vs_hbm,
           idxs, idxd, krows, qrows, vrows, gsem, wsem):
        c = lax.axis_index("c")
        s = lax.axis_index("s")
        base0 = (s * 2 + c) * per_w

        def dstep(g, _):
            for b in (0, 1):
                i = 2 * g + b
                base = pl.multiple_of(base0 + i * CH, 8)

                @pl.when(g > 0)
                def _():
                    pltpu.make_async_copy(
                        krows[b], ks_hbm.at[pl.ds(base, CH)],
                        wsem[3 * b]).wait()
                    pltpu.make_async_copy(
                        qrows[b], qd_hbm.at[pl.ds(base, CH)],
                        wsem[3 * b + 1]).wait()
                    pltpu.make_async_copy(
                        vrows[b], vs_hbm.at[pl.ds(base, CH)],
                        wsem[3 * b + 2]).wait()

                pltpu.sync_copy(src_hbm.at[pl.ds(base, CH)], idxs[b])
                pltpu.sync_copy(dst_hbm.at[pl.ds(base, CH)], idxd[b])
                pltpu.async_copy(k_hbm.at[idxs[b]], krows[b], gsem[3 * b])
                pltpu.async_copy(q_hbm.at[idxd[b]], qrows[b],
                                 gsem[3 * b + 1])
                pltpu.async_copy(v_hbm.at[idxs[b]], vrows[b],
                                 gsem[3 * b + 2])
            for b in (0, 1):
                i = 2 * g + b
                base = pl.multiple_of(base0 + i * CH, 8)
                pltpu.make_async_copy(k_hbm.at[idxs[b]], krows[b],
                                      gsem[3 * b]).wait()
                pltpu.async_copy(krows[b], ks_hbm.at[pl.ds(base, CH)],
                                 wsem[3 * b])
                pltpu.make_async_copy(q_hbm.at[idxd[b]], qrows[b],
                                      gsem[3 * b + 1]).wait()
                pltpu.async_copy(qrows[b], qd_hbm.at[pl.ds(base, CH)],
                                 wsem[3 * b + 1])
                pltpu.make_async_copy(v_hbm.at[idxs[b]], vrows[b],
                                      gsem[3 * b + 2]).wait()
                pltpu.async_copy(vrows[b], vs_hbm.at[pl.ds(base, CH)],
                                 wsem[3 * b + 2])
            return 0

        lax.fori_loop(0, nfull // 2, dstep, 0)
        for b in (0, 1):
            base = pl.multiple_of(base0, 8)
            pltpu.make_async_copy(krows[b], ks_hbm.at[pl.ds(base, CH)],
                                  wsem[3 * b]).wait()
            pltpu.make_async_copy(qrows[b], qd_hbm.at[pl.ds(base, CH)],
                                  wsem[3 * b + 1]).wait()
            pltpu.make_async_copy(vrows[b], vs_hbm.at[pl.ds(base, CH)],
                                  wsem[3 * b + 2]).wait()
        tb = pl.multiple_of(base0 + nfull * CH, 8)
        pltpu.sync_copy(src_hbm.at[pl.ds(tb, tail)],
                        idxs[0].at[pl.ds(0, tail)])
        pltpu.sync_copy(dst_hbm.at[pl.ds(tb, tail)],
                        idxd[0].at[pl.ds(0, tail)])
        pltpu.async_copy(k_hbm.at[idxs[0].at[pl.ds(0, tail)]],
                         krows[0].at[pl.ds(0, tail)], gsem[0]).wait()
        pltpu.sync_copy(krows[0].at[pl.ds(0, tail)],
                        ks_hbm.at[pl.ds(tb, tail)])
        pltpu.async_copy(q_hbm.at[idxd[0].at[pl.ds(0, tail)]],
                         qrows[0].at[pl.ds(0, tail)], gsem[1]).wait()
        pltpu.sync_copy(qrows[0].at[pl.ds(0, tail)],
                        qd_hbm.at[pl.ds(tb, tail)])
        pltpu.async_copy(v_hbm.at[idxs[0].at[pl.ds(0, tail)]],
                         vrows[0].at[pl.ds(0, tail)], gsem[2]).wait()
        pltpu.sync_copy(vrows[0].at[pl.ds(0, tail)],
                        vs_hbm.at[pl.ds(tb, tail)])

    return kk(ktab, qtab, vtab, src, dst)


_WIN = 640
_ZR = 40


def _win_start(s):
    return pl.multiple_of(
        jnp.minimum(s * (_N // 16) - lax.rem(s, 8), _N - _WIN), 8)


def _zero_rows(zbuf, acc, start):
    def zrow(i, _):
        def zlane(j, _):
            zbuf[i, pl.ds(j * 16, 16)] = jnp.zeros((16,), jnp.float32)
            return 0
        lax.fori_loop(0, 8, zlane, 0)
        return 0

    lax.fori_loop(0, _ZR, zrow, 0)

    def zcp(i, _):
        off = pl.multiple_of(start + i * _ZR, 8)
        pltpu.sync_copy(zbuf, acc.at[pl.ds(off, _ZR)])
        return 0

    lax.fori_loop(0, _WIN // _ZR, zcp, 0)


def _scatter_wv(wv, dst):
    per_s = _E // 16
    CH = 80
    nchunk = per_s // CH
    mesh = plsc.VectorSubcoreMesh(core_axis_name="c", subcore_axis_name="s",
                                  num_cores=2, num_subcores=16)

    @functools.partial(
        pl.kernel,
        out_type=jax.ShapeDtypeStruct((_N, _D), jnp.float32),
        mesh=mesh,
        scratch_types=[
            pltpu.VMEM_SHARED((_N, 128), jnp.float32),
            pltpu.VMEM((_ZR, 128), jnp.float32),
            [pltpu.VMEM((CH, 128), jnp.float32)] * 2,
            [pltpu.VMEM((CH,), jnp.int32)] * 2,
            [pltpu.SemaphoreType.DMA] * 2,
            [pltpu.SemaphoreType.DMA] * 2,
            [pltpu.SemaphoreType.DMA] * 2,
        ])
    def kk(wv_hbm, dst_hbm, out_hbm, acc, zbuf, buf, idxb, psem, isem, ssem):
        c = lax.axis_index("c")
        s = lax.axis_index("s")
        coff = c * 128
        start = _win_start(s)
        _zero_rows(zbuf, acc, start)
        plsc.subcore_barrier()
        base0 = s * per_s

        def dstep(g, _):
            for b in (0, 1):
                i = 2 * g + b
                base = pl.multiple_of(base0 + i * CH, 8)

                @pl.when(g > 0)
                def _():
                    pltpu.make_async_copy(buf[b], acc.at[idxb[b]],
                                          ssem[b]).wait()

                pltpu.async_copy(dst_hbm.at[pl.ds(base, CH)], idxb[b],
                                 isem[b])
                pltpu.async_copy(
                    wv_hbm.at[pl.ds(base, CH), pl.ds(coff, 128)],
                    buf[b], psem[b])
            for b in (0, 1):
                i = 2 * g + b
                base = pl.multiple_of(base0 + i * CH, 8)
                pltpu.make_async_copy(dst_hbm.at[pl.ds(base, CH)], idxb[b],
                                      isem[b]).wait()
                pltpu.make_async_copy(
                    wv_hbm.at[pl.ds(base, CH), pl.ds(coff, 128)],
                    buf[b], psem[b]).wait()
                pltpu.async_copy(buf[b], acc.at[idxb[b]], ssem[b], add=True)
            return 0

        lax.fori_loop(0, nchunk // 2, dstep, 0)
        lb = pl.multiple_of(base0 + (nchunk - 1) * CH, 8)
        pltpu.make_async_copy(buf[0], acc.at[idxb[0]], ssem[0]).wait()
        pltpu.sync_copy(dst_hbm.at[pl.ds(lb, CH)], idxb[0])
        pltpu.sync_copy(wv_hbm.at[pl.ds(lb, CH), pl.ds(coff, 128)], buf[0])
        pltpu.async_copy(buf[0], acc.at[idxb[0]], ssem[0], add=True)
        pltpu.make_async_copy(buf[0], acc.at[idxb[0]], ssem[0]).wait()
        pltpu.make_async_copy(buf[1], acc.at[idxb[1]], ssem[1]).wait()
        plsc.subcore_barrier()
        pltpu.sync_copy(acc.at[pl.ds(start, _WIN)],
                        out_hbm.at[pl.ds(start, _WIN), pl.ds(coff, 128)])

    return kk(wv, dst)


def _scatter_z(sexp128, dst):
    half = _E // 2
    per_s = half // 16
    CH = 40
    nchunk = per_s // CH
    mesh = plsc.VectorSubcoreMesh(core_axis_name="c", subcore_axis_name="s",
                                  num_cores=2, num_subcores=16)

    @functools.partial(
        pl.kernel,
        out_type=[jax.ShapeDtypeStruct((_N, 128), jnp.float32),
                  jax.ShapeDtypeStruct((_N, 128), jnp.float32)],
        mesh=mesh,
        scratch_types=[
            pltpu.VMEM_SHARED((_N, 128), jnp.float32),
            pltpu.VMEM((_ZR, 128), jnp.float32),
            [pltpu.VMEM((CH, 128), jnp.float32)] * 2,
            [pltpu.VMEM((CH,), jnp.int32)] * 2,
            [pltpu.SemaphoreType.DMA] * 2,
            [pltpu.SemaphoreType.DMA] * 2,
            [pltpu.SemaphoreType.DMA] * 2,
        ])
    def kk(se_hbm, dst_hbm, outa_hbm, outb_hbm, acc, zbuf, buf, idxb,
           psem, isem, ssem):
        c = lax.axis_index("c")
        s = lax.axis_index("s")
        start = _win_start(s)
        _zero_rows(zbuf, acc, start)
        plsc.subcore_barrier()
        base0 = c * half + s * per_s

        def dstep(g, _):
            for b in (0, 1):
                i = 2 * g + b
                base = pl.multiple_of(base0 + i * CH, 8)

                @pl.when(g > 0)
                def _():
                    pltpu.make_async_copy(buf[b], acc.at[idxb[b]],
                                          ssem[b]).wait()

                pltpu.async_copy(dst_hbm.at[pl.ds(base, CH)], idxb[b],
                                 isem[b])
                pltpu.async_copy(se_hbm.at[pl.ds(base, CH)], buf[b],
                                 psem[b])
            for b in (0, 1):
                i = 2 * g + b
                base = pl.multiple_of(base0 + i * CH, 8)
                pltpu.make_async_copy(dst_hbm.at[pl.ds(base, CH)], idxb[b],
                                      isem[b]).wait()
                pltpu.make_async_copy(se_hbm.at[pl.ds(base, CH)], buf[b],
                                      psem[b]).wait()
                pltpu.async_copy(buf[b], acc.at[idxb[b]], ssem[b], add=True)
            return 0

        lax.fori_loop(0, nchunk // 2, dstep, 0)
        lb = pl.multiple_of(base0 + (nchunk - 1) * CH, 8)
        pltpu.make_async_copy(buf[0], acc.at[idxb[0]], ssem[0]).wait()
        pltpu.sync_copy(dst_hbm.at[pl.ds(lb, CH)], idxb[0])
        pltpu.sync_copy(se_hbm.at[pl.ds(lb, CH)], buf[0])
        pltpu.async_copy(buf[0], acc.at[idxb[0]], ssem[0], add=True)
        pltpu.make_async_copy(buf[0], acc.at[idxb[0]], ssem[0]).wait()
        pltpu.make_async_copy(buf[1], acc.at[idxb[1]], ssem[1]).wait()
        plsc.subcore_barrier()

        @pl.when(c == 0)
        def _():
            pltpu.sync_copy(acc.at[pl.ds(start, _WIN)],
                            outa_hbm.at[pl.ds(start, _WIN)])

        @pl.when(c == 1)
        def _():
            pltpu.sync_copy(acc.at[pl.ds(start, _WIN)],
                            outb_hbm.at[pl.ds(start, _WIN)])

    return kk(sexp128, dst)


def kernel(h, e, edge_index, Wq, Wk, Wv, We, Wo_h, bo_h, Wo_e, bo_e,
           W1h, b1h, W2h, b2h, W1e, b1e, W2e, b2e):
    src = edge_index[0].astype(jnp.int32)
    dst = edge_index[1].astype(jnp.int32)
    heads = jnp.arange(_D, dtype=jnp.int32) // _DH
    hsum = (heads[:, None] == jnp.arange(_H, dtype=jnp.int32)[None, :]
            ).astype(jnp.float32)
    hexp = hsum.T
    j8 = jnp.arange(128, dtype=jnp.int32) // 8
    hrep = (jnp.arange(_H, dtype=jnp.int32)[:, None] == j8[None, :]
            ).astype(jnp.float32)
    zsel = (j8[:, None] == heads[None, :]).astype(jnp.float32) * (1.0 / 8.0)
    bo_h2 = bo_h.reshape(1, _D)
    bo_e2 = bo_e.reshape(1, _D)

    q, k, v = _qkv(h, Wq, Wk, Wv)
    ksrc, qdst, vsrc = _gather3(k, q, v, src, dst)
    eo, wv, sexp128, stats_e = _edge1(e, ksrc, qdst, vsrc, We, Wo_e, bo_e2,
                                      hsum, hexp, hrep)
    wV = _scatter_wv(wv, dst)
    za, zb = _scatter_z(sexp128, dst)
    ho, stats_h = _node1(wV, za, zb, h, Wo_h, bo_h2, zsel)
    pre2_h, stats2_h = _ffn(ho, stats_h, _N, W1h, b1h.reshape(1, 2 * _D),
                            W2h, b2h.reshape(1, _D), 1000)
    pre2_e, stats2_e = _ffn(eo, stats_e, _E, W1e, b1e.reshape(1, 2 * _D),
                            W2e, b2e.reshape(1, _D), 1600)
    hh = _norm(pre2_h, stats2_h, _N, 1000)
    ee = _norm(pre2_e, stats2_e, _E, 1600)
    return (hh, ee)


# speedup vs baseline: 21.3403x; 1.0018x over previous
import functools

import jax
import jax.numpy as jnp
from jax import lax
from jax.experimental import pallas as pl
from jax.experimental.pallas import tpu as pltpu
from jax.experimental.pallas import tpu_sc as plsc

_N = 10000
_E = 160000
_D = 256
_H = 16
_DH = 16


def _dotbf(a, b):
    return jnp.dot(a.astype(jnp.bfloat16), b.astype(jnp.bfloat16),
                   preferred_element_type=jnp.float32)


def _qkv_body(h_ref, wq_ref, wk_ref, wv_ref, q_ref, k_ref, v_ref):
    hb = h_ref[...]
    q_ref[...] = _dotbf(hb, wq_ref[...])
    k_ref[...] = _dotbf(hb, wk_ref[...])
    v_ref[...] = _dotbf(hb, wv_ref[...])


def _qkv(h, Wq, Wk, Wv):
    BN = 1000
    mat = pl.BlockSpec((_D, _D), lambda i: (0, 0))
    blk = pl.BlockSpec((BN, _D), lambda i: (i, 0))
    return pl.pallas_call(
        _qkv_body,
        grid=(_N // BN,),
        in_specs=[blk, mat, mat, mat],
        out_specs=[blk, blk, blk],
        out_shape=[jax.ShapeDtypeStruct((_N, _D), jnp.float32)] * 3,
    )(h, Wq, Wk, Wv)


def _edge1_body(e_ref, ks_ref, qd_ref, vs_ref, we_ref, woe_ref, boe_ref,
                hsum_ref, hexp_ref, hrep_ref, eo_ref, wv_ref, sexp_ref,
                stats_ref):
    i = pl.program_id(0)
    eb = e_ref[...]
    pe = _dotbf(eb, we_ref[...])
    score = ks_ref[...] * qd_ref[...] * (pe * (1.0 / 4.0))
    hs = jnp.dot(score, hsum_ref[...], preferred_element_type=jnp.float32)
    sexp = jnp.exp(jnp.clip(hs, -5.0, 5.0))
    sexp_ref[...] = jnp.dot(sexp, hrep_ref[...],
                            preferred_element_type=jnp.float32)
    eo = eb + _dotbf(score, woe_ref[...]) + boe_ref[...]
    eo_ref[...] = eo
    wv_ref[...] = vs_ref[...] * jnp.dot(sexp, hexp_ref[...],
                                        preferred_element_type=jnp.float32)

    @pl.when(i == 0)
    def _():
        stats_ref[...] = jnp.zeros_like(stats_ref)

    stats_ref[0:1, :] += jnp.sum(eo, axis=0, keepdims=True)
    stats_ref[1:2, :] += jnp.sum(eo * eo, axis=0, keepdims=True)


def _edge1(e, ksrc, qdst, vsrc, We, Wo_e, bo_e, hsum, hexp, hrep):
    BE = 1600
    blk = pl.BlockSpec((BE, _D), lambda i: (i, 0))
    mat = pl.BlockSpec((_D, _D), lambda i: (0, 0))
    return pl.pallas_call(
        _edge1_body,
        grid=(_E // BE,),
        in_specs=[blk, blk, blk, blk, mat, mat,
                  pl.BlockSpec((1, _D), lambda i: (0, 0)),
                  pl.BlockSpec((_D, _H), lambda i: (0, 0)),
                  pl.BlockSpec((_H, _D), lambda i: (0, 0)),
                  pl.BlockSpec((_H, 128), lambda i: (0, 0))],
        out_specs=[blk, blk,
                   pl.BlockSpec((BE, 128), lambda i: (i, 0)),
                   pl.BlockSpec((8, _D), lambda i: (0, 0))],
        out_shape=[jax.ShapeDtypeStruct((_E, _D), jnp.float32),
                   jax.ShapeDtypeStruct((_E, _D), jnp.float32),
                   jax.ShapeDtypeStruct((_E, 128), jnp.float32),
                   jax.ShapeDtypeStruct((8, _D), jnp.float32)],
    )(e, ksrc, qdst, vsrc, We, Wo_e, bo_e, hsum, hexp, hrep)


def _node1_body(wv_ref, za_ref, zb_ref, h_ref, wo_ref, bo_ref, zsel_ref,
                ho_ref, stats_ref):
    i = pl.program_id(0)
    zrep = jnp.dot(za_ref[...] + zb_ref[...], zsel_ref[...],
                   preferred_element_type=jnp.float32)
    hattn = wv_ref[...] / (zrep + 1e-6)
    ho = h_ref[...] + _dotbf(hattn, wo_ref[...]) + bo_ref[...]
    ho_ref[...] = ho

    @pl.when(i == 0)
    def _():
        stats_ref[...] = jnp.zeros_like(stats_ref)

    stats_ref[0:1, :] += jnp.sum(ho, axis=0, keepdims=True)
    stats_ref[1:2, :] += jnp.sum(ho * ho, axis=0, keepdims=True)


def _node1(wV, za, zb, h, Wo_h, bo_h, zsel):
    BN = 1000
    blk = pl.BlockSpec((BN, _D), lambda i: (i, 0))
    zblk = pl.BlockSpec((BN, 128), lambda i: (i, 0))
    return pl.pallas_call(
        _node1_body,
        grid=(_N // BN,),
        in_specs=[blk, zblk, zblk, blk,
                  pl.BlockSpec((_D, _D), lambda i: (0, 0)),
                  pl.BlockSpec((1, _D), lambda i: (0, 0)),
                  pl.BlockSpec((128, _D), lambda i: (0, 0))],
        out_specs=[blk, pl.BlockSpec((8, _D), lambda i: (0, 0))],
        out_shape=[jax.ShapeDtypeStruct((_N, _D), jnp.float32),
                   jax.ShapeDtypeStruct((8, _D), jnp.float32)],
    )(wV, za, zb, h, Wo_h, bo_h, zsel)


def _ffn_body(count, x_ref, stats_ref, w1_ref, b1_ref, w2_ref, b2_ref,
              out_ref, stats2_ref):
    i = pl.program_id(0)
    mu = stats_ref[0:1, :] * (1.0 / count)
    var = stats_ref[1:2, :] * (1.0 / count) - mu * mu
    inv = lax.rsqrt(var + 1e-5)
    xb = (x_ref[...] - mu) * inv
    u = jnp.maximum(_dotbf(xb, w1_ref[...]) + b1_ref[...], 0.0)
    pre2 = xb + _dotbf(u, w2_ref[...]) + b2_ref[...]
    out_ref[...] = pre2

    @pl.when(i == 0)
    def _():
        stats2_ref[...] = jnp.zeros_like(stats2_ref)

    stats2_ref[0:1, :] += jnp.sum(pre2, axis=0, keepdims=True)
    stats2_ref[1:2, :] += jnp.sum(pre2 * pre2, axis=0, keepdims=True)


def _ffn(x, stats, count, W1, b1, W2, b2, bx):
    rows = x.shape[0]
    blk = pl.BlockSpec((bx, _D), lambda i: (i, 0))
    return pl.pallas_call(
        functools.partial(_ffn_body, float(count)),
        grid=(rows // bx,),
        in_specs=[blk, pl.BlockSpec((8, _D), lambda i: (0, 0)),
                  pl.BlockSpec((_D, 2 * _D), lambda i: (0, 0)),
                  pl.BlockSpec((1, 2 * _D), lambda i: (0, 0)),
                  pl.BlockSpec((2 * _D, _D), lambda i: (0, 0)),
                  pl.BlockSpec((1, _D), lambda i: (0, 0))],
        out_specs=[blk, pl.BlockSpec((8, _D), lambda i: (0, 0))],
        out_shape=[jax.ShapeDtypeStruct((rows, _D), jnp.float32),
                   jax.ShapeDtypeStruct((8, _D), jnp.float32)],
    )(x, stats, W1, b1, W2, b2)


def _norm_body(count, x_ref, stats_ref, out_ref):
    mu = stats_ref[0:1, :] * (1.0 / count)
    var = stats_ref[1:2, :] * (1.0 / count) - mu * mu
    inv = lax.rsqrt(var + 1e-5)
    out_ref[...] = (x_ref[...] - mu) * inv


def _norm(x, stats, count, bx):
    rows = x.shape[0]
    blk = pl.BlockSpec((bx, _D), lambda i: (i, 0))
    return pl.pallas_call(
        functools.partial(_norm_body, float(count)),
        grid=(rows // bx,),
        in_specs=[blk, pl.BlockSpec((8, _D), lambda i: (0, 0))],
        out_specs=blk,
        out_shape=jax.ShapeDtypeStruct((rows, _D), jnp.float32),
    )(x, stats)


def _gather3(ktab, qtab, vtab, src, dst):
    NW = 32
    per_w = _E // NW
    CH = 64
    nfull = per_w // CH
    tail = per_w - nfull * CH
    mesh = plsc.VectorSubcoreMesh(core_axis_name="c", subcore_axis_name="s",
                                  num_cores=2, num_subcores=16)

    @functools.partial(
        pl.kernel,
        out_type=[jax.ShapeDtypeStruct((_E, _D), jnp.float32)] * 3,
        mesh=mesh,
        scratch_types=[
            [pltpu.VMEM((CH,), jnp.int32)] * 2,
            [pltpu.VMEM((CH,), jnp.int32)] * 2,
            [pltpu.VMEM((CH, _D), jnp.float32)] * 2,
            [pltpu.VMEM((CH, _D), jnp.float32)] * 2,
            [pltpu.VMEM((CH, _D), jnp.float32)] * 2,
            [pltpu.SemaphoreType.DMA] * 6,
            [pltpu.SemaphoreType.DMA] * 6,
        ])
    def kk(k_hbm, q_hbm, v_hbm, src_hbm, dst_hbm, ks_hbm, qd_hbm, vs_hbm,
           idxs, idxd, krows, qrows, vrows, gsem, wsem):
        c = lax.axis_index("c")
        s = lax.axis_index("s")
        base0 = (s * 2 + c) * per_w

        def dstep(g, _):
            for b in (0, 1):
                i = 2 * g + b
                base = pl.multiple_of(base0 + i * CH, 8)

                @pl.when(g > 0)
                def _():
                    pltpu.make_async_copy(
                        krows[b], ks_hbm.at[pl.ds(base, CH)],
                        wsem[3 * b]).wait()
                    pltpu.make_async_copy(
                        qrows[b], qd_hbm.at[pl.ds(base, CH)],
                        wsem[3 * b + 1]).wait()
                    pltpu.make_async_copy(
                        vrows[b], vs_hbm.at[pl.ds(base, CH)],
                        wsem[3 * b + 2]).wait()

                pltpu.sync_copy(src_hbm.at[pl.ds(base, CH)], idxs[b])
                pltpu.sync_copy(dst_hbm.at[pl.ds(base, CH)], idxd[b])
                pltpu.async_copy(k_hbm.at[idxs[b]], krows[b], gsem[3 * b])
                pltpu.async_copy(q_hbm.at[idxd[b]], qrows[b],
                                 gsem[3 * b + 1])
                pltpu.async_copy(v_hbm.at[idxs[b]], vrows[b],
                                 gsem[3 * b + 2])
            for b in (0, 1):
                i = 2 * g + b
                base = pl.multiple_of(base0 + i * CH, 8)
                pltpu.make_async_copy(k_hbm.at[idxs[b]], krows[b],
                                      gsem[3 * b]).wait()
                pltpu.async_copy(krows[b], ks_hbm.at[pl.ds(base, CH)],
                                 wsem[3 * b])
                pltpu.make_async_copy(q_hbm.at[idxd[b]], qrows[b],
                                      gsem[3 * b + 1]).wait()
                pltpu.async_copy(qrows[b], qd_hbm.at[pl.ds(base, CH)],
                                 wsem[3 * b + 1])
                pltpu.make_async_copy(v_hbm.at[idxs[b]], vrows[b],
                                      gsem[3 * b + 2]).wait()
                pltpu.async_copy(vrows[b], vs_hbm.at[pl.ds(base, CH)],
                                 wsem[3 * b + 2])
            return 0

        lax.fori_loop(0, nfull // 2, dstep, 0)
        for b in (0, 1):
            base = pl.multiple_of(base0, 8)
            pltpu.make_async_copy(krows[b], ks_hbm.at[pl.ds(base, CH)],
                                  wsem[3 * b]).wait()
            pltpu.make_async_copy(qrows[b], qd_hbm.at[pl.ds(base, CH)],
                                  wsem[3 * b + 1]).wait()
            pltpu.make_async_copy(vrows[b], vs_hbm.at[pl.ds(base, CH)],
                                  wsem[3 * b + 2]).wait()
        tb = pl.multiple_of(base0 + nfull * CH, 8)
        pltpu.sync_copy(src_hbm.at[pl.ds(tb, tail)],
                        idxs[0].at[pl.ds(0, tail)])
        pltpu.sync_copy(dst_hbm.at[pl.ds(tb, tail)],
                        idxd[0].at[pl.ds(0, tail)])
        pltpu.async_copy(k_hbm.at[idxs[0].at[pl.ds(0, tail)]],
                         krows[0].at[pl.ds(0, tail)], gsem[0]).wait()
        pltpu.sync_copy(krows[0].at[pl.ds(0, tail)],
                        ks_hbm.at[pl.ds(tb, tail)])
        pltpu.async_copy(q_hbm.at[idxd[0].at[pl.ds(0, tail)]],
                         qrows[0].at[pl.ds(0, tail)], gsem[1]).wait()
        pltpu.sync_copy(qrows[0].at[pl.ds(0, tail)],
                        qd_hbm.at[pl.ds(tb, tail)])
        pltpu.async_copy(v_hbm.at[idxs[0].at[pl.ds(0, tail)]],
                         vrows[0].at[pl.ds(0, tail)], gsem[2]).wait()
        pltpu.sync_copy(vrows[0].at[pl.ds(0, tail)],
                        vs_hbm.at[pl.ds(tb, tail)])

    return kk(ktab, qtab, vtab, src, dst)


_WIN = 640
_ZR = 40


def _win_start(s):
    return pl.multiple_of(
        jnp.minimum(s * (_N // 16) - lax.rem(s, 8), _N - _WIN), 8)


def _zero_rows(zbuf, acc, start):
    def zrow(i, _):
        def zlane(j, _):
            zbuf[i, pl.ds(j * 16, 16)] = jnp.zeros((16,), jnp.float32)
            return 0
        lax.fori_loop(0, 8, zlane, 0)
        return 0

    lax.fori_loop(0, _ZR, zrow, 0)

    def zcp(i, _):
        off = pl.multiple_of(start + i * _ZR, 8)
        pltpu.sync_copy(zbuf, acc.at[pl.ds(off, _ZR)])
        return 0

    lax.fori_loop(0, _WIN // _ZR, zcp, 0)


def _scatter_wv(wv, dst):
    per_s = _E // 16
    CH = 80
    nchunk = per_s // CH
    mesh = plsc.VectorSubcoreMesh(core_axis_name="c", subcore_axis_name="s",
                                  num_cores=2, num_subcores=16)

    @functools.partial(
        pl.kernel,
        out_type=jax.ShapeDtypeStruct((_N, _D), jnp.float32),
        mesh=mesh,
        scratch_types=[
            pltpu.VMEM_SHARED((_N, 128), jnp.float32),
            pltpu.VMEM((_ZR, 128), jnp.float32),
            [pltpu.VMEM((CH, 128), jnp.float32)] * 2,
            [pltpu.VMEM((CH,), jnp.int32)] * 2,
            [pltpu.SemaphoreType.DMA] * 2,
            [pltpu.SemaphoreType.DMA] * 2,
            [pltpu.SemaphoreType.DMA] * 2,
        ])
    def kk(wv_hbm, dst_hbm, out_hbm, acc, zbuf, buf, idxb, psem, isem, ssem):
        c = lax.axis_index("c")
        s = lax.axis_index("s")
        coff = c * 128
        start = _win_start(s)
        _zero_rows(zbuf, acc, start)
        plsc.subcore_barrier()
        base0 = s * per_s

        def dstep(g, _):
            for b in (0, 1):
                i = 2 * g + b
                base = pl.multiple_of(base0 + i * CH, 8)

                @pl.when(g > 0)
                def _():
                    pltpu.make_async_copy(buf[b], acc.at[idxb[b]],
                                          ssem[b]).wait()

                pltpu.async_copy(dst_hbm.at[pl.ds(base, CH)], idxb[b],
                                 isem[b])
                pltpu.async_copy(
                    wv_hbm.at[pl.ds(base, CH), pl.ds(coff, 128)],
                    buf[b], psem[b])
            for b in (0, 1):
                i = 2 * g + b
                base = pl.multiple_of(base0 + i * CH, 8)
                pltpu.make_async_copy(dst_hbm.at[pl.ds(base, CH)], idxb[b],
                                      isem[b]).wait()
                pltpu.make_async_copy(
                    wv_hbm.at[pl.ds(base, CH), pl.ds(coff, 128)],
                    buf[b], psem[b]).wait()
                pltpu.async_copy(buf[b], acc.at[idxb[b]], ssem[b], add=True)
            return 0

        lax.fori_loop(0, nchunk // 2, dstep, 0)
        lb = pl.multiple_of(base0 + (nchunk - 1) * CH, 8)
        pltpu.make_async_copy(buf[0], acc.at[idxb[0]], ssem[0]).wait()
        pltpu.sync_copy(dst_hbm.at[pl.ds(lb, CH)], idxb[0])
        pltpu.sync_copy(wv_hbm.at[pl.ds(lb, CH), pl.ds(coff, 128)], buf[0])
        pltpu.async_copy(buf[0], acc.at[idxb[0]], ssem[0], add=True)
        pltpu.make_async_copy(buf[0], acc.at[idxb[0]], ssem[0]).wait()
        pltpu.make_async_copy(buf[1], acc.at[idxb[1]], ssem[1]).wait()
        plsc.subcore_barrier()
        pltpu.sync_copy(acc.at[pl.ds(start, _WIN)],
                        out_hbm.at[pl.ds(start, _WIN), pl.ds(coff, 128)])

    return kk(wv, dst)


def _scatter_z(sexp128, dst):
    half = _E // 2
    per_s = half // 16
    CH = 40
    nchunk = per_s // CH
    mesh = plsc.VectorSubcoreMesh(core_axis_name="c", subcore_axis_name="s",
                                  num_cores=2, num_subcores=16)

    @functools.partial(
        pl.kernel,
        out_type=[jax.ShapeDtypeStruct((_N, 128), jnp.float32),
                  jax.ShapeDtypeStruct((_N, 128), jnp.float32)],
        mesh=mesh,
        scratch_types=[
            pltpu.VMEM_SHARED((_N, 128), jnp.float32),
            pltpu.VMEM((_ZR, 128), jnp.float32),
            [pltpu.VMEM((CH, 128), jnp.float32)] * 2,
            [pltpu.VMEM((CH,), jnp.int32)] * 2,
            [pltpu.SemaphoreType.DMA] * 2,
            [pltpu.SemaphoreType.DMA] * 2,
            [pltpu.SemaphoreType.DMA] * 2,
        ])
    def kk(se_hbm, dst_hbm, outa_hbm, outb_hbm, acc, zbuf, buf, idxb,
           psem, isem, ssem):
        c = lax.axis_index("c")
        s = lax.axis_index("s")
        start = _win_start(s)
        _zero_rows(zbuf, acc, start)
        plsc.subcore_barrier()
        base0 = c * half + s * per_s

        def dstep(g, _):
            for b in (0, 1):
                i = 2 * g + b
                base = pl.multiple_of(base0 + i * CH, 8)

                @pl.when(g > 0)
                def _():
                    pltpu.make_async_copy(buf[b], acc.at[idxb[b]],
                                          ssem[b]).wait()

                pltpu.async_copy(dst_hbm.at[pl.ds(base, CH)], idxb[b],
                                 isem[b])
                pltpu.async_copy(se_hbm.at[pl.ds(base, CH)], buf[b],
                                 psem[b])
            for b in (0, 1):
                i = 2 * g + b
                base = pl.multiple_of(base0 + i * CH, 8)
                pltpu.make_async_copy(dst_hbm.at[pl.ds(base, CH)], idxb[b],
                                      isem[b]).wait()
                pltpu.make_async_copy(se_hbm.at[pl.ds(base, CH)], buf[b],
                                      psem[b]).wait()
                pltpu.async_copy(buf[b], acc.at[idxb[b]], ssem[b], add=True)
            return 0

        lax.fori_loop(0, nchunk // 2, dstep, 0)
        lb = pl.multiple_of(base0 + (nchunk - 1) * CH, 8)
        pltpu.make_async_copy(buf[0], acc.at[idxb[0]], ssem[0]).wait()
        pltpu.sync_copy(dst_hbm.at[pl.ds(lb, CH)], idxb[0])
        pltpu.sync_copy(se_hbm.at[pl.ds(lb, CH)], buf[0])
        pltpu.async_copy(buf[0], acc.at[idxb[0]], ssem[0], add=True)
        pltpu.make_async_copy(buf[0], acc.at[idxb[0]], ssem[0]).wait()
        pltpu.make_async_copy(buf[1], acc.at[idxb[1]], ssem[1]).wait()
        plsc.subcore_barrier()

        @pl.when(c == 0)
        def _():
            pltpu.sync_copy(acc.at[pl.ds(start, _WIN)],
                            outa_hbm.at[pl.ds(start, _WIN)])

        @pl.when(c == 1)
        def _():
            pltpu.sync_copy(acc.at[pl.ds(start, _WIN)],
                            outb_hbm.at[pl.ds(start, _WIN)])

    return kk(sexp128, dst)


def kernel(h, e, edge_index, Wq, Wk, Wv, We, Wo_h, bo_h, Wo_e, bo_e,
           W1h, b1h, W2h, b2h, W1e, b1e, W2e, b2e):
    src = edge_index[0].astype(jnp.int32)
    dst = edge_index[1].astype(jnp.int32)
    heads = jnp.arange(_D, dtype=jnp.int32) // _DH
    hsum = (heads[:, None] == jnp.arange(_H, dtype=jnp.int32)[None, :]
            ).astype(jnp.float32)
    hexp = hsum.T
    j8 = jnp.arange(128, dtype=jnp.int32) // 8
    hrep = (jnp.arange(_H, dtype=jnp.int32)[:, None] == j8[None, :]
            ).astype(jnp.float32)
    zsel = (j8[:, None] == heads[None, :]).astype(jnp.float32) * (1.0 / 8.0)
    bo_h2 = bo_h.reshape(1, _D)
    bo_e2 = bo_e.reshape(1, _D)

    q, k, v = _qkv(h, Wq, Wk, Wv)
    ksrc, qdst, vsrc = _gather3(k, q, v, src, dst)
    eo, wv, sexp128, stats_e = _edge1(e, ksrc, qdst, vsrc, We, Wo_e, bo_e2,
                                      hsum, hexp, hrep)
    wV = _scatter_wv(wv, dst)
    za, zb = _scatter_z(sexp128, dst)
    ho, stats_h = _node1(wV, za, zb, h, Wo_h, bo_h2, zsel)
    pre2_h, stats2_h = _ffn(ho, stats_h, _N, W1h, b1h.reshape(1, 2 * _D),
                            W2h, b2h.reshape(1, _D), 1000)
    pre2_e, stats2_e = _ffn(eo, stats_e, _E, W1e, b1e.reshape(1, 2 * _D),
                            W2e, b2e.reshape(1, _D), 1600)
    hh = _norm(pre2_h, stats2_h, _N, 1000)
    ee = _norm(pre2_e, stats2_e, _E, 1600)
    return (hh, ee)


# speedup vs baseline: 21.4077x; 1.0032x over previous
import functools

import jax
import jax.numpy as jnp
from jax import lax
from jax.experimental import pallas as pl
from jax.experimental.pallas import tpu as pltpu
from jax.experimental.pallas import tpu_sc as plsc

_N = 10000
_E = 160000
_D = 256
_H = 16
_DH = 16


def _dotbf(a, b):
    return jnp.dot(a.astype(jnp.bfloat16), b.astype(jnp.bfloat16),
                   preferred_element_type=jnp.float32)


def _qkv_body(h_ref, wq_ref, wk_ref, wv_ref, q_ref, kv_ref):
    hb = h_ref[...]
    q_ref[...] = _dotbf(hb, wq_ref[...])
    kv_ref[:, 0:_D] = _dotbf(hb, wk_ref[...])
    kv_ref[:, _D:2 * _D] = _dotbf(hb, wv_ref[...])


def _qkv(h, Wq, Wk, Wv):
    BN = 1000
    mat = pl.BlockSpec((_D, _D), lambda i: (0, 0))
    blk = pl.BlockSpec((BN, _D), lambda i: (i, 0))
    return pl.pallas_call(
        _qkv_body,
        grid=(_N // BN,),
        in_specs=[blk, mat, mat, mat],
        out_specs=[blk, pl.BlockSpec((BN, 2 * _D), lambda i: (i, 0))],
        out_shape=[jax.ShapeDtypeStruct((_N, _D), jnp.float32),
                   jax.ShapeDtypeStruct((_N, 2 * _D), jnp.float32)],
    )(h, Wq, Wk, Wv)


def _edge1_body(e_ref, kvs_ref, qd_ref, we_ref, woe_ref, boe_ref,
                hsum_ref, hexp_ref, hrep_ref, eo_ref, wv_ref, sexp_ref,
                stats_ref):
    i = pl.program_id(0)
    eb = e_ref[...]
    pe = _dotbf(eb, we_ref[...])
    score = kvs_ref[:, 0:_D] * qd_ref[...] * (pe * (1.0 / 4.0))
    hs = jnp.dot(score, hsum_ref[...], preferred_element_type=jnp.float32)
    sexp = jnp.exp(jnp.clip(hs, -5.0, 5.0))
    sexp_ref[...] = jnp.dot(sexp, hrep_ref[...],
                            preferred_element_type=jnp.float32)
    eo = eb + _dotbf(score, woe_ref[...]) + boe_ref[...]
    eo_ref[...] = eo
    wv_ref[...] = kvs_ref[:, _D:2 * _D] * jnp.dot(
        sexp, hexp_ref[...], preferred_element_type=jnp.float32)

    @pl.when(i == 0)
    def _():
        stats_ref[...] = jnp.zeros_like(stats_ref)

    stats_ref[0:1, :] += jnp.sum(eo, axis=0, keepdims=True)
    stats_ref[1:2, :] += jnp.sum(eo * eo, axis=0, keepdims=True)


def _edge1(e, kvsrc, qdst, We, Wo_e, bo_e, hsum, hexp, hrep):
    BE = 1600
    blk = pl.BlockSpec((BE, _D), lambda i: (i, 0))
    mat = pl.BlockSpec((_D, _D), lambda i: (0, 0))
    return pl.pallas_call(
        _edge1_body,
        grid=(_E // BE,),
        in_specs=[blk, pl.BlockSpec((BE, 2 * _D), lambda i: (i, 0)),
                  blk, mat, mat,
                  pl.BlockSpec((1, _D), lambda i: (0, 0)),
                  pl.BlockSpec((_D, _H), lambda i: (0, 0)),
                  pl.BlockSpec((_H, _D), lambda i: (0, 0)),
                  pl.BlockSpec((_H, 128), lambda i: (0, 0))],
        out_specs=[blk, blk,
                   pl.BlockSpec((BE, 128), lambda i: (i, 0)),
                   pl.BlockSpec((8, _D), lambda i: (0, 0))],
        out_shape=[jax.ShapeDtypeStruct((_E, _D), jnp.float32),
                   jax.ShapeDtypeStruct((_E, _D), jnp.float32),
                   jax.ShapeDtypeStruct((_E, 128), jnp.float32),
                   jax.ShapeDtypeStruct((8, _D), jnp.float32)],
    )(e, kvsrc, qdst, We, Wo_e, bo_e, hsum, hexp, hrep)


def _node1_body(wv_ref, za_ref, zb_ref, h_ref, wo_ref, bo_ref, zsel_ref,
                ho_ref, stats_ref):
    i = pl.program_id(0)
    zrep = jnp.dot(za_ref[...] + zb_ref[...], zsel_ref[...],
                   preferred_element_type=jnp.float32)
    hattn = wv_ref[...] / (zrep + 1e-6)
    ho = h_ref[...] + _dotbf(hattn, wo_ref[...]) + bo_ref[...]
    ho_ref[...] = ho

    @pl.when(i == 0)
    def _():
        stats_ref[...] = jnp.zeros_like(stats_ref)

    stats_ref[0:1, :] += jnp.sum(ho, axis=0, keepdims=True)
    stats_ref[1:2, :] += jnp.sum(ho * ho, axis=0, keepdims=True)


def _node1(wV, za, zb, h, Wo_h, bo_h, zsel):
    BN = 1000
    blk = pl.BlockSpec((BN, _D), lambda i: (i, 0))
    zblk = pl.BlockSpec((BN, 128), lambda i: (i, 0))
    return pl.pallas_call(
        _node1_body,
        grid=(_N // BN,),
        in_specs=[blk, zblk, zblk, blk,
                  pl.BlockSpec((_D, _D), lambda i: (0, 0)),
                  pl.BlockSpec((1, _D), lambda i: (0, 0)),
                  pl.BlockSpec((128, _D), lambda i: (0, 0))],
        out_specs=[blk, pl.BlockSpec((8, _D), lambda i: (0, 0))],
        out_shape=[jax.ShapeDtypeStruct((_N, _D), jnp.float32),
                   jax.ShapeDtypeStruct((8, _D), jnp.float32)],
    )(wV, za, zb, h, Wo_h, bo_h, zsel)


def _ffn_body(count, x_ref, stats_ref, w1_ref, b1_ref, w2_ref, b2_ref,
              out_ref, stats2_ref):
    i = pl.program_id(0)
    mu = stats_ref[0:1, :] * (1.0 / count)
    var = stats_ref[1:2, :] * (1.0 / count) - mu * mu
    inv = lax.rsqrt(var + 1e-5)
    xb = (x_ref[...] - mu) * inv
    u = jnp.maximum(_dotbf(xb, w1_ref[...]) + b1_ref[...], 0.0)
    pre2 = xb + _dotbf(u, w2_ref[...]) + b2_ref[...]
    out_ref[...] = pre2

    @pl.when(i == 0)
    def _():
        stats2_ref[...] = jnp.zeros_like(stats2_ref)

    stats2_ref[0:1, :] += jnp.sum(pre2, axis=0, keepdims=True)
    stats2_ref[1:2, :] += jnp.sum(pre2 * pre2, axis=0, keepdims=True)


def _ffn(x, stats, count, W1, b1, W2, b2, bx):
    rows = x.shape[0]
    blk = pl.BlockSpec((bx, _D), lambda i: (i, 0))
    return pl.pallas_call(
        functools.partial(_ffn_body, float(count)),
        grid=(rows // bx,),
        in_specs=[blk, pl.BlockSpec((8, _D), lambda i: (0, 0)),
                  pl.BlockSpec((_D, 2 * _D), lambda i: (0, 0)),
                  pl.BlockSpec((1, 2 * _D), lambda i: (0, 0)),
                  pl.BlockSpec((2 * _D, _D), lambda i: (0, 0)),
                  pl.BlockSpec((1, _D), lambda i: (0, 0))],
        out_specs=[blk, pl.BlockSpec((8, _D), lambda i: (0, 0))],
        out_shape=[jax.ShapeDtypeStruct((rows, _D), jnp.float32),
                   jax.ShapeDtypeStruct((8, _D), jnp.float32)],
    )(x, stats, W1, b1, W2, b2)


def _norm_body(count, x_ref, stats_ref, out_ref):
    mu = stats_ref[0:1, :] * (1.0 / count)
    var = stats_ref[1:2, :] * (1.0 / count) - mu * mu
    inv = lax.rsqrt(var + 1e-5)
    out_ref[...] = (x_ref[...] - mu) * inv


def _norm(x, stats, count, bx):
    rows = x.shape[0]
    blk = pl.BlockSpec((bx, _D), lambda i: (i, 0))
    return pl.pallas_call(
        functools.partial(_norm_body, float(count)),
        grid=(rows // bx,),
        in_specs=[blk, pl.BlockSpec((8, _D), lambda i: (0, 0))],
        out_specs=blk,
        out_shape=jax.ShapeDtypeStruct((rows, _D), jnp.float32),
    )(x, stats)


def _gather3(ktab, qtab, src, dst):
    NW = 32
    per_w = _E // NW
    CH = 64
    nfull = per_w // CH
    tail = per_w - nfull * CH
    mesh = plsc.VectorSubcoreMesh(core_axis_name="c", subcore_axis_name="s",
                                  num_cores=2, num_subcores=16)

    @functools.partial(
        pl.kernel,
        out_type=[jax.ShapeDtypeStruct((_E, 2 * _D), jnp.float32),
                  jax.ShapeDtypeStruct((_E, _D), jnp.float32)],
        mesh=mesh,
        scratch_types=[
            [pltpu.VMEM((CH,), jnp.int32)] * 2,
            [pltpu.VMEM((CH,), jnp.int32)] * 2,
            [pltpu.VMEM((CH, 2 * _D), jnp.float32)] * 2,
            [pltpu.VMEM((CH, _D), jnp.float32)] * 2,
            [pltpu.SemaphoreType.DMA] * 4,
            [pltpu.SemaphoreType.DMA] * 4,
        ])
    def kk(kv_hbm, q_hbm, src_hbm, dst_hbm, kvs_hbm, qd_hbm,
           idxs, idxd, kvrows, qrows, gsem, wsem):
        c = lax.axis_index("c")
        s = lax.axis_index("s")
        base0 = (s * 2 + c) * per_w

        def dstep(g, _):
            for b in (0, 1):
                i = 2 * g + b
                base = pl.multiple_of(base0 + i * CH, 8)

                @pl.when(g > 0)
                def _():
                    pltpu.make_async_copy(
                        kvrows[b], kvs_hbm.at[pl.ds(base, CH)],
                        wsem[2 * b]).wait()
                    pltpu.make_async_copy(
                        qrows[b], qd_hbm.at[pl.ds(base, CH)],
                        wsem[2 * b + 1]).wait()

                pltpu.sync_copy(src_hbm.at[pl.ds(base, CH)], idxs[b])
                pltpu.sync_copy(dst_hbm.at[pl.ds(base, CH)], idxd[b])
                pltpu.async_copy(kv_hbm.at[idxs[b]], kvrows[b], gsem[2 * b])
                pltpu.async_copy(q_hbm.at[idxd[b]], qrows[b],
                                 gsem[2 * b + 1])
            for b in (0, 1):
                i = 2 * g + b
                base = pl.multiple_of(base0 + i * CH, 8)
                pltpu.make_async_copy(kv_hbm.at[idxs[b]], kvrows[b],
                                      gsem[2 * b]).wait()
                pltpu.async_copy(kvrows[b], kvs_hbm.at[pl.ds(base, CH)],
                                 wsem[2 * b])
                pltpu.make_async_copy(q_hbm.at[idxd[b]], qrows[b],
                                      gsem[2 * b + 1]).wait()
                pltpu.async_copy(qrows[b], qd_hbm.at[pl.ds(base, CH)],
                                 wsem[2 * b + 1])
            return 0

        lax.fori_loop(0, nfull // 2, dstep, 0)
        for b in (0, 1):
            base = pl.multiple_of(base0, 8)
            pltpu.make_async_copy(kvrows[b], kvs_hbm.at[pl.ds(base, CH)],
                                  wsem[2 * b]).wait()
            pltpu.make_async_copy(qrows[b], qd_hbm.at[pl.ds(base, CH)],
                                  wsem[2 * b + 1]).wait()
        tb = pl.multiple_of(base0 + nfull * CH, 8)
        pltpu.sync_copy(src_hbm.at[pl.ds(tb, tail)],
                        idxs[0].at[pl.ds(0, tail)])
        pltpu.sync_copy(dst_hbm.at[pl.ds(tb, tail)],
                        idxd[0].at[pl.ds(0, tail)])
        pltpu.async_copy(kv_hbm.at[idxs[0].at[pl.ds(0, tail)]],
                         kvrows[0].at[pl.ds(0, tail)], gsem[0]).wait()
        pltpu.sync_copy(kvrows[0].at[pl.ds(0, tail)],
                        kvs_hbm.at[pl.ds(tb, tail)])
        pltpu.async_copy(q_hbm.at[idxd[0].at[pl.ds(0, tail)]],
                         qrows[0].at[pl.ds(0, tail)], gsem[1]).wait()
        pltpu.sync_copy(qrows[0].at[pl.ds(0, tail)],
                        qd_hbm.at[pl.ds(tb, tail)])

    return kk(ktab, qtab, src, dst)


_WIN = 640
_ZR = 40


def _win_start(s):
    return pl.multiple_of(
        jnp.minimum(s * (_N // 16) - lax.rem(s, 8), _N - _WIN), 8)


def _zero_rows(zbuf, acc, start):
    def zrow(i, _):
        def zlane(j, _):
            zbuf[i, pl.ds(j * 16, 16)] = jnp.zeros((16,), jnp.float32)
            return 0
        lax.fori_loop(0, 8, zlane, 0)
        return 0

    lax.fori_loop(0, _ZR, zrow, 0)

    def zcp(i, _):
        off = pl.multiple_of(start + i * _ZR, 8)
        pltpu.sync_copy(zbuf, acc.at[pl.ds(off, _ZR)])
        return 0

    lax.fori_loop(0, _WIN // _ZR, zcp, 0)


def _scatter_wv(wv, dst):
    per_s = _E // 16
    CH = 80
    nchunk = per_s // CH
    mesh = plsc.VectorSubcoreMesh(core_axis_name="c", subcore_axis_name="s",
                                  num_cores=2, num_subcores=16)

    @functools.partial(
        pl.kernel,
        out_type=jax.ShapeDtypeStruct((_N, _D), jnp.float32),
        mesh=mesh,
        scratch_types=[
            pltpu.VMEM_SHARED((_N, 128), jnp.float32),
            pltpu.VMEM((_ZR, 128), jnp.float32),
            [pltpu.VMEM((CH, 128), jnp.float32)] * 2,
            [pltpu.VMEM((CH,), jnp.int32)] * 2,
            [pltpu.SemaphoreType.DMA] * 2,
            [pltpu.SemaphoreType.DMA] * 2,
            [pltpu.SemaphoreType.DMA] * 2,
        ])
    def kk(wv_hbm, dst_hbm, out_hbm, acc, zbuf, buf, idxb, psem, isem, ssem):
        c = lax.axis_index("c")
        s = lax.axis_index("s")
        coff = c * 128
        start = _win_start(s)
        _zero_rows(zbuf, acc, start)
        plsc.subcore_barrier()
        base0 = s * per_s

        def dstep(g, _):
            for b in (0, 1):
                i = 2 * g + b
                base = pl.multiple_of(base0 + i * CH, 8)

                @pl.when(g > 0)
                def _():
                    pltpu.make_async_copy(buf[b], acc.at[idxb[b]],
                                          ssem[b]).wait()

                pltpu.async_copy(dst_hbm.at[pl.ds(base, CH)], idxb[b],
                                 isem[b])
                pltpu.async_copy(
                    wv_hbm.at[pl.ds(base, CH), pl.ds(coff, 128)],
                    buf[b], psem[b])
            for b in (0, 1):
                i = 2 * g + b
                base = pl.multiple_of(base0 + i * CH, 8)
                pltpu.make_async_copy(dst_hbm.at[pl.ds(base, CH)], idxb[b],
                                      isem[b]).wait()
                pltpu.make_async_copy(
                    wv_hbm.at[pl.ds(base, CH), pl.ds(coff, 128)],
                    buf[b], psem[b]).wait()
                pltpu.async_copy(buf[b], acc.at[idxb[b]], ssem[b], add=True)
            return 0

        lax.fori_loop(0, nchunk // 2, dstep, 0)
        lb = pl.multiple_of(base0 + (nchunk - 1) * CH, 8)
        pltpu.make_async_copy(buf[0], acc.at[idxb[0]], ssem[0]).wait()
        pltpu.sync_copy(dst_hbm.at[pl.ds(lb, CH)], idxb[0])
        pltpu.sync_copy(wv_hbm.at[pl.ds(lb, CH), pl.ds(coff, 128)], buf[0])
        pltpu.async_copy(buf[0], acc.at[idxb[0]], ssem[0], add=True)
        pltpu.make_async_copy(buf[0], acc.at[idxb[0]], ssem[0]).wait()
        pltpu.make_async_copy(buf[1], acc.at[idxb[1]], ssem[1]).wait()
        plsc.subcore_barrier()
        pltpu.sync_copy(acc.at[pl.ds(start, _WIN)],
                        out_hbm.at[pl.ds(start, _WIN), pl.ds(coff, 128)])

    return kk(wv, dst)


def _scatter_z(sexp128, dst):
    half = _E // 2
    per_s = half // 16
    CH = 40
    nchunk = per_s // CH
    mesh = plsc.VectorSubcoreMesh(core_axis_name="c", subcore_axis_name="s",
                                  num_cores=2, num_subcores=16)

    @functools.partial(
        pl.kernel,
        out_type=[jax.ShapeDtypeStruct((_N, 128), jnp.float32),
                  jax.ShapeDtypeStruct((_N, 128), jnp.float32)],
        mesh=mesh,
        scratch_types=[
            pltpu.VMEM_SHARED((_N, 128), jnp.float32),
            pltpu.VMEM((_ZR, 128), jnp.float32),
            [pltpu.VMEM((CH, 128), jnp.float32)] * 2,
            [pltpu.VMEM((CH,), jnp.int32)] * 2,
            [pltpu.SemaphoreType.DMA] * 2,
            [pltpu.SemaphoreType.DMA] * 2,
            [pltpu.SemaphoreType.DMA] * 2,
        ])
    def kk(se_hbm, dst_hbm, outa_hbm, outb_hbm, acc, zbuf, buf, idxb,
           psem, isem, ssem):
        c = lax.axis_index("c")
        s = lax.axis_index("s")
        start = _win_start(s)
        _zero_rows(zbuf, acc, start)
        plsc.subcore_barrier()
        base0 = c * half + s * per_s

        def dstep(g, _):
            for b in (0, 1):
                i = 2 * g + b
                base = pl.multiple_of(base0 + i * CH, 8)

                @pl.when(g > 0)
                def _():
                    pltpu.make_async_copy(buf[b], acc.at[idxb[b]],
                                          ssem[b]).wait()

                pltpu.async_copy(dst_hbm.at[pl.ds(base, CH)], idxb[b],
                                 isem[b])
                pltpu.async_copy(se_hbm.at[pl.ds(base, CH)], buf[b],
                                 psem[b])
            for b in (0, 1):
                i = 2 * g + b
                base = pl.multiple_of(base0 + i * CH, 8)
                pltpu.make_async_copy(dst_hbm.at[pl.ds(base, CH)], idxb[b],
                                      isem[b]).wait()
                pltpu.make_async_copy(se_hbm.at[pl.ds(base, CH)], buf[b],
                                      psem[b]).wait()
                pltpu.async_copy(buf[b], acc.at[idxb[b]], ssem[b], add=True)
            return 0

        lax.fori_loop(0, nchunk // 2, dstep, 0)
        lb = pl.multiple_of(base0 + (nchunk - 1) * CH, 8)
        pltpu.make_async_copy(buf[0], acc.at[idxb[0]], ssem[0]).wait()
        pltpu.sync_copy(dst_hbm.at[pl.ds(lb, CH)], idxb[0])
        pltpu.sync_copy(se_hbm.at[pl.ds(lb, CH)], buf[0])
        pltpu.async_copy(buf[0], acc.at[idxb[0]], ssem[0], add=True)
        pltpu.make_async_copy(buf[0], acc.at[idxb[0]], ssem[0]).wait()
        pltpu.make_async_copy(buf[1], acc.at[idxb[1]], ssem[1]).wait()
        plsc.subcore_barrier()

        @pl.when(c == 0)
        def _():
            pltpu.sync_copy(acc.at[pl.ds(start, _WIN)],
                            outa_hbm.at[pl.ds(start, _WIN)])

        @pl.when(c == 1)
        def _():
            pltpu.sync_copy(acc.at[pl.ds(start, _WIN)],
                            outb_hbm.at[pl.ds(start, _WIN)])

    return kk(sexp128, dst)


def kernel(h, e, edge_index, Wq, Wk, Wv, We, Wo_h, bo_h, Wo_e, bo_e,
           W1h, b1h, W2h, b2h, W1e, b1e, W2e, b2e):
    src = edge_index[0].astype(jnp.int32)
    dst = edge_index[1].astype(jnp.int32)
    heads = jnp.arange(_D, dtype=jnp.int32) // _DH
    hsum = (heads[:, None] == jnp.arange(_H, dtype=jnp.int32)[None, :]
            ).astype(jnp.float32)
    hexp = hsum.T
    j8 = jnp.arange(128, dtype=jnp.int32) // 8
    hrep = (jnp.arange(_H, dtype=jnp.int32)[:, None] == j8[None, :]
            ).astype(jnp.float32)
    zsel = (j8[:, None] == heads[None, :]).astype(jnp.float32) * (1.0 / 8.0)
    bo_h2 = bo_h.reshape(1, _D)
    bo_e2 = bo_e.reshape(1, _D)

    q, kv = _qkv(h, Wq, Wk, Wv)
    kvsrc, qdst = _gather3(kv, q, src, dst)
    eo, wv, sexp128, stats_e = _edge1(e, kvsrc, qdst, We, Wo_e, bo_e2,
                                      hsum, hexp, hrep)
    wV = _scatter_wv(wv, dst)
    za, zb = _scatter_z(sexp128, dst)
    ho, stats_h = _node1(wV, za, zb, h, Wo_h, bo_h2, zsel)
    pre2_h, stats2_h = _ffn(ho, stats_h, _N, W1h, b1h.reshape(1, 2 * _D),
                            W2h, b2h.reshape(1, _D), 1000)
    pre2_e, stats2_e = _ffn(eo, stats_e, _E, W1e, b1e.reshape(1, 2 * _D),
                            W2e, b2e.reshape(1, _D), 1600)
    hh = _norm(pre2_h, stats2_h, _N, 1000)
    ee = _norm(pre2_e, stats2_e, _E, 1600)
    return (hh, ee)


# speedup vs baseline: 21.4493x; 1.0019x over previous
import functools

import jax
import jax.numpy as jnp
from jax import lax
from jax.experimental import pallas as pl
from jax.experimental.pallas import tpu as pltpu
from jax.experimental.pallas import tpu_sc as plsc

_N = 10000
_E = 160000
_D = 256
_H = 16
_DH = 16


def _dotbf(a, b):
    return jnp.dot(a, b, preferred_element_type=jnp.float32)


def _qkv_body(h_ref, wq_ref, wk_ref, wv_ref, q_ref, kv_ref):
    hb = h_ref[...]
    q_ref[...] = _dotbf(hb, wq_ref[...])
    kv_ref[:, 0:_D] = _dotbf(hb, wk_ref[...])
    kv_ref[:, _D:2 * _D] = _dotbf(hb, wv_ref[...])


def _qkv(h, Wq, Wk, Wv):
    BN = 1000
    mat = pl.BlockSpec((_D, _D), lambda i: (0, 0))
    blk = pl.BlockSpec((BN, _D), lambda i: (i, 0))
    return pl.pallas_call(
        _qkv_body,
        grid=(_N // BN,),
        in_specs=[blk, mat, mat, mat],
        out_specs=[blk, pl.BlockSpec((BN, 2 * _D), lambda i: (i, 0))],
        out_shape=[jax.ShapeDtypeStruct((_N, _D), jnp.float32),
                   jax.ShapeDtypeStruct((_N, 2 * _D), jnp.float32)],
    )(h, Wq, Wk, Wv)


def _edge1_body(e_ref, kvs_ref, qd_ref, we_ref, woe_ref, boe_ref,
                hsum_ref, hexp_ref, hrep_ref, eo_ref, wv_ref, sexp_ref,
                stats_ref):
    i = pl.program_id(0)
    eb = e_ref[...]
    pe = _dotbf(eb, we_ref[...])
    score = kvs_ref[:, 0:_D] * qd_ref[...] * (pe * (1.0 / 4.0))
    hs = jnp.dot(score, hsum_ref[...], preferred_element_type=jnp.float32)
    sexp = jnp.exp(jnp.clip(hs, -5.0, 5.0))
    sexp_ref[...] = jnp.dot(sexp, hrep_ref[...],
                            preferred_element_type=jnp.float32)
    eo = eb + _dotbf(score, woe_ref[...]) + boe_ref[...]
    eo_ref[...] = eo
    wv_ref[...] = kvs_ref[:, _D:2 * _D] * jnp.dot(
        sexp, hexp_ref[...], preferred_element_type=jnp.float32)

    @pl.when(i == 0)
    def _():
        stats_ref[...] = jnp.zeros_like(stats_ref)

    stats_ref[0:1, :] += jnp.sum(eo, axis=0, keepdims=True)
    stats_ref[1:2, :] += jnp.sum(eo * eo, axis=0, keepdims=True)


def _edge1(e, kvsrc, qdst, We, Wo_e, bo_e, hsum, hexp, hrep):
    BE = 1600
    blk = pl.BlockSpec((BE, _D), lambda i: (i, 0))
    mat = pl.BlockSpec((_D, _D), lambda i: (0, 0))
    return pl.pallas_call(
        _edge1_body,
        grid=(_E // BE,),
        in_specs=[blk, pl.BlockSpec((BE, 2 * _D), lambda i: (i, 0)),
                  blk, mat, mat,
                  pl.BlockSpec((1, _D), lambda i: (0, 0)),
                  pl.BlockSpec((_D, _H), lambda i: (0, 0)),
                  pl.BlockSpec((_H, _D), lambda i: (0, 0)),
                  pl.BlockSpec((_H, 128), lambda i: (0, 0))],
        out_specs=[blk, blk,
                   pl.BlockSpec((BE, 128), lambda i: (i, 0)),
                   pl.BlockSpec((8, _D), lambda i: (0, 0))],
        out_shape=[jax.ShapeDtypeStruct((_E, _D), jnp.float32),
                   jax.ShapeDtypeStruct((_E, _D), jnp.float32),
                   jax.ShapeDtypeStruct((_E, 128), jnp.float32),
                   jax.ShapeDtypeStruct((8, _D), jnp.float32)],
    )(e, kvsrc, qdst, We, Wo_e, bo_e, hsum, hexp, hrep)


def _node1_body(wv_ref, za_ref, zb_ref, h_ref, wo_ref, bo_ref, zsel_ref,
                ho_ref, stats_ref):
    i = pl.program_id(0)
    zrep = jnp.dot(za_ref[...] + zb_ref[...], zsel_ref[...],
                   preferred_element_type=jnp.float32)
    hattn = wv_ref[...] / (zrep + 1e-6)
    ho = h_ref[...] + _dotbf(hattn, wo_ref[...]) + bo_ref[...]
    ho_ref[...] = ho

    @pl.when(i == 0)
    def _():
        stats_ref[...] = jnp.zeros_like(stats_ref)

    stats_ref[0:1, :] += jnp.sum(ho, axis=0, keepdims=True)
    stats_ref[1:2, :] += jnp.sum(ho * ho, axis=0, keepdims=True)


def _node1(wV, za, zb, h, Wo_h, bo_h, zsel):
    BN = 1000
    blk = pl.BlockSpec((BN, _D), lambda i: (i, 0))
    zblk = pl.BlockSpec((BN, 128), lambda i: (i, 0))
    return pl.pallas_call(
        _node1_body,
        grid=(_N // BN,),
        in_specs=[blk, zblk, zblk, blk,
                  pl.BlockSpec((_D, _D), lambda i: (0, 0)),
                  pl.BlockSpec((1, _D), lambda i: (0, 0)),
                  pl.BlockSpec((128, _D), lambda i: (0, 0))],
        out_specs=[blk, pl.BlockSpec((8, _D), lambda i: (0, 0))],
        out_shape=[jax.ShapeDtypeStruct((_N, _D), jnp.float32),
                   jax.ShapeDtypeStruct((8, _D), jnp.float32)],
    )(wV, za, zb, h, Wo_h, bo_h, zsel)


def _ffn_body(count, x_ref, stats_ref, w1_ref, b1_ref, w2_ref, b2_ref,
              out_ref, stats2_ref):
    i = pl.program_id(0)
    mu = stats_ref[0:1, :] * (1.0 / count)
    var = stats_ref[1:2, :] * (1.0 / count) - mu * mu
    inv = lax.rsqrt(var + 1e-5)
    xb = (x_ref[...] - mu) * inv
    u = jnp.maximum(_dotbf(xb, w1_ref[...]) + b1_ref[...], 0.0)
    pre2 = xb + _dotbf(u, w2_ref[...]) + b2_ref[...]
    out_ref[...] = pre2

    @pl.when(i == 0)
    def _():
        stats2_ref[...] = jnp.zeros_like(stats2_ref)

    stats2_ref[0:1, :] += jnp.sum(pre2, axis=0, keepdims=True)
    stats2_ref[1:2, :] += jnp.sum(pre2 * pre2, axis=0, keepdims=True)


def _ffn(x, stats, count, W1, b1, W2, b2, bx):
    rows = x.shape[0]
    blk = pl.BlockSpec((bx, _D), lambda i: (i, 0))
    return pl.pallas_call(
        functools.partial(_ffn_body, float(count)),
        grid=(rows // bx,),
        in_specs=[blk, pl.BlockSpec((8, _D), lambda i: (0, 0)),
                  pl.BlockSpec((_D, 2 * _D), lambda i: (0, 0)),
                  pl.BlockSpec((1, 2 * _D), lambda i: (0, 0)),
                  pl.BlockSpec((2 * _D, _D), lambda i: (0, 0)),
                  pl.BlockSpec((1, _D), lambda i: (0, 0))],
        out_specs=[blk, pl.BlockSpec((8, _D), lambda i: (0, 0))],
        out_shape=[jax.ShapeDtypeStruct((rows, _D), jnp.float32),
                   jax.ShapeDtypeStruct((8, _D), jnp.float32)],
    )(x, stats, W1, b1, W2, b2)


def _norm_body(count, x_ref, stats_ref, out_ref):
    mu = stats_ref[0:1, :] * (1.0 / count)
    var = stats_ref[1:2, :] * (1.0 / count) - mu * mu
    inv = lax.rsqrt(var + 1e-5)
    out_ref[...] = (x_ref[...] - mu) * inv


def _norm(x, stats, count, bx):
    rows = x.shape[0]
    blk = pl.BlockSpec((bx, _D), lambda i: (i, 0))
    return pl.pallas_call(
        functools.partial(_norm_body, float(count)),
        grid=(rows // bx,),
        in_specs=[blk, pl.BlockSpec((8, _D), lambda i: (0, 0))],
        out_specs=blk,
        out_shape=jax.ShapeDtypeStruct((rows, _D), jnp.float32),
    )(x, stats)


def _gather3(ktab, qtab, src, dst):
    NW = 32
    per_w = _E // NW
    CH = 64
    nfull = per_w // CH
    tail = per_w - nfull * CH
    mesh = plsc.VectorSubcoreMesh(core_axis_name="c", subcore_axis_name="s",
                                  num_cores=2, num_subcores=16)

    @functools.partial(
        pl.kernel,
        out_type=[jax.ShapeDtypeStruct((_E, 2 * _D), jnp.float32),
                  jax.ShapeDtypeStruct((_E, _D), jnp.float32)],
        mesh=mesh,
        scratch_types=[
            [pltpu.VMEM((CH,), jnp.int32)] * 2,
            [pltpu.VMEM((CH,), jnp.int32)] * 2,
            [pltpu.VMEM((CH, 2 * _D), jnp.float32)] * 2,
            [pltpu.VMEM((CH, _D), jnp.float32)] * 2,
            [pltpu.SemaphoreType.DMA] * 4,
            [pltpu.SemaphoreType.DMA] * 4,
        ])
    def kk(kv_hbm, q_hbm, src_hbm, dst_hbm, kvs_hbm, qd_hbm,
           idxs, idxd, kvrows, qrows, gsem, wsem):
        c = lax.axis_index("c")
        s = lax.axis_index("s")
        base0 = (s * 2 + c) * per_w

        def dstep(g, _):
            for b in (0, 1):
                i = 2 * g + b
                base = pl.multiple_of(base0 + i * CH, 8)

                @pl.when(g > 0)
                def _():
                    pltpu.make_async_copy(
                        kvrows[b], kvs_hbm.at[pl.ds(base, CH)],
                        wsem[2 * b]).wait()
                    pltpu.make_async_copy(
                        qrows[b], qd_hbm.at[pl.ds(base, CH)],
                        wsem[2 * b + 1]).wait()

                pltpu.sync_copy(src_hbm.at[pl.ds(base, CH)], idxs[b])
                pltpu.sync_copy(dst_hbm.at[pl.ds(base, CH)], idxd[b])
                pltpu.async_copy(kv_hbm.at[idxs[b]], kvrows[b], gsem[2 * b])
                pltpu.async_copy(q_hbm.at[idxd[b]], qrows[b],
                                 gsem[2 * b + 1])
            for b in (0, 1):
                i = 2 * g + b
                base = pl.multiple_of(base0 + i * CH, 8)
                pltpu.make_async_copy(kv_hbm.at[idxs[b]], kvrows[b],
                                      gsem[2 * b]).wait()
                pltpu.async_copy(kvrows[b], kvs_hbm.at[pl.ds(base, CH)],
                                 wsem[2 * b])
                pltpu.make_async_copy(q_hbm.at[idxd[b]], qrows[b],
                                      gsem[2 * b + 1]).wait()
                pltpu.async_copy(qrows[b], qd_hbm.at[pl.ds(base, CH)],
                                 wsem[2 * b + 1])
            return 0

        lax.fori_loop(0, nfull // 2, dstep, 0)
        for b in (0, 1):
            base = pl.multiple_of(base0, 8)
            pltpu.make_async_copy(kvrows[b], kvs_hbm.at[pl.ds(base, CH)],
                                  wsem[2 * b]).wait()
            pltpu.make_async_copy(qrows[b], qd_hbm.at[pl.ds(base, CH)],
                                  wsem[2 * b + 1]).wait()
        tb = pl.multiple_of(base0 + nfull * CH, 8)
        pltpu.sync_copy(src_hbm.at[pl.ds(tb, tail)],
                        idxs[0].at[pl.ds(0, tail)])
        pltpu.sync_copy(dst_hbm.at[pl.ds(tb, tail)],
                        idxd[0].at[pl.ds(0, tail)])
        pltpu.async_copy(kv_hbm.at[idxs[0].at[pl.ds(0, tail)]],
                         kvrows[0].at[pl.ds(0, tail)], gsem[0]).wait()
        pltpu.sync_copy(kvrows[0].at[pl.ds(0, tail)],
                        kvs_hbm.at[pl.ds(tb, tail)])
        pltpu.async_copy(q_hbm.at[idxd[0].at[pl.ds(0, tail)]],
                         qrows[0].at[pl.ds(0, tail)], gsem[1]).wait()
        pltpu.sync_copy(qrows[0].at[pl.ds(0, tail)],
                        qd_hbm.at[pl.ds(tb, tail)])

    return kk(ktab, qtab, src, dst)


_WIN = 640
_ZR = 40


def _win_start(s):
    return pl.multiple_of(
        jnp.minimum(s * (_N // 16) - lax.rem(s, 8), _N - _WIN), 8)


def _zero_rows(zbuf, acc, start):
    def zrow(i, _):
        def zlane(j, _):
            zbuf[i, pl.ds(j * 16, 16)] = jnp.zeros((16,), jnp.float32)
            return 0
        lax.fori_loop(0, 8, zlane, 0)
        return 0

    lax.fori_loop(0, _ZR, zrow, 0)

    def zcp(i, _):
        off = pl.multiple_of(start + i * _ZR, 8)
        pltpu.sync_copy(zbuf, acc.at[pl.ds(off, _ZR)])
        return 0

    lax.fori_loop(0, _WIN // _ZR, zcp, 0)


def _scatter_wv(wv, dst):
    per_s = _E // 16
    CH = 80
    nchunk = per_s // CH
    mesh = plsc.VectorSubcoreMesh(core_axis_name="c", subcore_axis_name="s",
                                  num_cores=2, num_subcores=16)

    @functools.partial(
        pl.kernel,
        out_type=jax.ShapeDtypeStruct((_N, _D), jnp.float32),
        mesh=mesh,
        scratch_types=[
            pltpu.VMEM_SHARED((_N, 128), jnp.float32),
            pltpu.VMEM((_ZR, 128), jnp.float32),
            [pltpu.VMEM((CH, 128), jnp.float32)] * 2,
            [pltpu.VMEM((CH,), jnp.int32)] * 2,
            [pltpu.SemaphoreType.DMA] * 2,
            [pltpu.SemaphoreType.DMA] * 2,
            [pltpu.SemaphoreType.DMA] * 2,
        ])
    def kk(wv_hbm, dst_hbm, out_hbm, acc, zbuf, buf, idxb, psem, isem, ssem):
        c = lax.axis_index("c")
        s = lax.axis_index("s")
        coff = c * 128
        start = _win_start(s)
        _zero_rows(zbuf, acc, start)
        plsc.subcore_barrier()
        base0 = s * per_s

        def dstep(g, _):
            for b in (0, 1):
                i = 2 * g + b
                base = pl.multiple_of(base0 + i * CH, 8)

                @pl.when(g > 0)
                def _():
                    pltpu.make_async_copy(buf[b], acc.at[idxb[b]],
                                          ssem[b]).wait()

                pltpu.async_copy(dst_hbm.at[pl.ds(base, CH)], idxb[b],
                                 isem[b])
                pltpu.async_copy(
                    wv_hbm.at[pl.ds(base, CH), pl.ds(coff, 128)],
                    buf[b], psem[b])
            for b in (0, 1):
                i = 2 * g + b
                base = pl.multiple_of(base0 + i * CH, 8)
                pltpu.make_async_copy(dst_hbm.at[pl.ds(base, CH)], idxb[b],
                                      isem[b]).wait()
                pltpu.make_async_copy(
                    wv_hbm.at[pl.ds(base, CH), pl.ds(coff, 128)],
                    buf[b], psem[b]).wait()
                pltpu.async_copy(buf[b], acc.at[idxb[b]], ssem[b], add=True)
            return 0

        lax.fori_loop(0, nchunk // 2, dstep, 0)
        lb = pl.multiple_of(base0 + (nchunk - 1) * CH, 8)
        pltpu.make_async_copy(buf[0], acc.at[idxb[0]], ssem[0]).wait()
        pltpu.sync_copy(dst_hbm.at[pl.ds(lb, CH)], idxb[0])
        pltpu.sync_copy(wv_hbm.at[pl.ds(lb, CH), pl.ds(coff, 128)], buf[0])
        pltpu.async_copy(buf[0], acc.at[idxb[0]], ssem[0], add=True)
        pltpu.make_async_copy(buf[0], acc.at[idxb[0]], ssem[0]).wait()
        pltpu.make_async_copy(buf[1], acc.at[idxb[1]], ssem[1]).wait()
        plsc.subcore_barrier()
        pltpu.sync_copy(acc.at[pl.ds(start, _WIN)],
                        out_hbm.at[pl.ds(start, _WIN), pl.ds(coff, 128)])

    return kk(wv, dst)


def _scatter_z(sexp128, dst):
    half = _E // 2
    per_s = half // 16
    CH = 40
    nchunk = per_s // CH
    mesh = plsc.VectorSubcoreMesh(core_axis_name="c", subcore_axis_name="s",
                                  num_cores=2, num_subcores=16)

    @functools.partial(
        pl.kernel,
        out_type=[jax.ShapeDtypeStruct((_N, 128), jnp.float32),
                  jax.ShapeDtypeStruct((_N, 128), jnp.float32)],
        mesh=mesh,
        scratch_types=[
            pltpu.VMEM_SHARED((_N, 128), jnp.float32),
            pltpu.VMEM((_ZR, 128), jnp.float32),
            [pltpu.VMEM((CH, 128), jnp.float32)] * 2,
            [pltpu.VMEM((CH,), jnp.int32)] * 2,
            [pltpu.SemaphoreType.DMA] * 2,
            [pltpu.SemaphoreType.DMA] * 2,
            [pltpu.SemaphoreType.DMA] * 2,
        ])
    def kk(se_hbm, dst_hbm, outa_hbm, outb_hbm, acc, zbuf, buf, idxb,
           psem, isem, ssem):
        c = lax.axis_index("c")
        s = lax.axis_index("s")
        start = _win_start(s)
        _zero_rows(zbuf, acc, start)
        plsc.subcore_barrier()
        base0 = c * half + s * per_s

        def dstep(g, _):
            for b in (0, 1):
                i = 2 * g + b
                base = pl.multiple_of(base0 + i * CH, 8)

                @pl.when(g > 0)
                def _():
                    pltpu.make_async_copy(buf[b], acc.at[idxb[b]],
                                          ssem[b]).wait()

                pltpu.async_copy(dst_hbm.at[pl.ds(base, CH)], idxb[b],
                                 isem[b])
                pltpu.async_copy(se_hbm.at[pl.ds(base, CH)], buf[b],
                                 psem[b])
            for b in (0, 1):
                i = 2 * g + b
                base = pl.multiple_of(base0 + i * CH, 8)
                pltpu.make_async_copy(dst_hbm.at[pl.ds(base, CH)], idxb[b],
                                      isem[b]).wait()
                pltpu.make_async_copy(se_hbm.at[pl.ds(base, CH)], buf[b],
                                      psem[b]).wait()
                pltpu.async_copy(buf[b], acc.at[idxb[b]], ssem[b], add=True)
            return 0

        lax.fori_loop(0, nchunk // 2, dstep, 0)
        lb = pl.multiple_of(base0 + (nchunk - 1) * CH, 8)
        pltpu.make_async_copy(buf[0], acc.at[idxb[0]], ssem[0]).wait()
        pltpu.sync_copy(dst_hbm.at[pl.ds(lb, CH)], idxb[0])
        pltpu.sync_copy(se_hbm.at[pl.ds(lb, CH)], buf[0])
        pltpu.async_copy(buf[0], acc.at[idxb[0]], ssem[0], add=True)
        pltpu.make_async_copy(buf[0], acc.at[idxb[0]], ssem[0]).wait()
        pltpu.make_async_copy(buf[1], acc.at[idxb[1]], ssem[1]).wait()
        plsc.subcore_barrier()

        @pl.when(c == 0)
        def _():
            pltpu.sync_copy(acc.at[pl.ds(start, _WIN)],
                            outa_hbm.at[pl.ds(start, _WIN)])

        @pl.when(c == 1)
        def _():
            pltpu.sync_copy(acc.at[pl.ds(start, _WIN)],
                            outb_hbm.at[pl.ds(start, _WIN)])

    return kk(sexp128, dst)


def kernel(h, e, edge_index, Wq, Wk, Wv, We, Wo_h, bo_h, Wo_e, bo_e,
           W1h, b1h, W2h, b2h, W1e, b1e, W2e, b2e):
    src = edge_index[0].astype(jnp.int32)
    dst = edge_index[1].astype(jnp.int32)
    heads = jnp.arange(_D, dtype=jnp.int32) // _DH
    hsum = (heads[:, None] == jnp.arange(_H, dtype=jnp.int32)[None, :]
            ).astype(jnp.float32)
    hexp = hsum.T
    j8 = jnp.arange(128, dtype=jnp.int32) // 8
    hrep = (jnp.arange(_H, dtype=jnp.int32)[:, None] == j8[None, :]
            ).astype(jnp.float32)
    zsel = (j8[:, None] == heads[None, :]).astype(jnp.float32) * (1.0 / 8.0)
    bo_h2 = bo_h.reshape(1, _D)
    bo_e2 = bo_e.reshape(1, _D)

    q, kv = _qkv(h, Wq, Wk, Wv)
    kvsrc, qdst = _gather3(kv, q, src, dst)
    eo, wv, sexp128, stats_e = _edge1(e, kvsrc, qdst, We, Wo_e, bo_e2,
                                      hsum, hexp, hrep)
    wV = _scatter_wv(wv, dst)
    za, zb = _scatter_z(sexp128, dst)
    ho, stats_h = _node1(wV, za, zb, h, Wo_h, bo_h2, zsel)
    pre2_h, stats2_h = _ffn(ho, stats_h, _N, W1h, b1h.reshape(1, 2 * _D),
                            W2h, b2h.reshape(1, _D), 1000)
    pre2_e, stats2_e = _ffn(eo, stats_e, _E, W1e, b1e.reshape(1, 2 * _D),
                            W2e, b2e.reshape(1, _D), 1600)
    hh = _norm(pre2_h, stats2_h, _N, 1000)
    ee = _norm(pre2_e, stats2_e, _E, 1600)
    return (hh, ee)
